# Initial kernel scaffold; baseline (speedup 1.0000x reference)
import jax
import jax.numpy as jnp
from jax.experimental import pallas as pl


def kernel(out_hm_rel, out_sub_offset, out_obj_offset, out_hm, out_wh, out_reg, tgt_hm_rel, tgt_offset, tgt_offset_mask, tgt_hm, tgt_reg_mask, tgt_ind, tgt_wh, tgt_reg):
    raise NotImplementedError("write your pallas kernel here")


# speedup vs baseline: 1.5830x; 1.5830x over previous
import functools

import jax
import jax.numpy as jnp
from jax import lax
from jax.experimental import pallas as pl
from jax.experimental.pallas import tpu as pltpu
from jax.experimental.pallas import tpu_sc as plsc

BSZ, KCLS, HH, WW = 2, 117, 128, 128
HWF = HH * WW
NTGT = 64
MAXO = 128
CHM = 80
ALPHA_ = 0.25
SCHUNK = HWF // NTGT


def _softplus(x):
    return jnp.maximum(x, 0.0) + jnp.log1p(jnp.exp(-jnp.abs(x)))


def _tc_body(ids_ref, cmap_ref, toff_ref, oot_ref, hr_ref, hm_ref, thm_ref,
             src_out, acc_out, srcs_scr, logit_scr, smatt_scr, acc_s):
    b = pl.program_id(0)
    t = pl.program_id(1)

    @pl.when((b == 0) & (t == 0))
    def _init_acc():
        acc_s[0] = 0.0
        acc_s[1] = 0.0
        acc_s[2] = 0.0
        acc_s[3] = 0.0

    @pl.when(t == 0)
    def _init_rows():
        srcs_scr[0:1, :] = jnp.zeros((1, 128), jnp.int32)
        logit_scr[0:1, :] = jnp.zeros((1, 128), jnp.float32)

    x = hr_ref[0]
    px = jax.nn.sigmoid(x)
    f0 = (1.0 - ALPHA_) * px * px * _softplus(x)
    s_f0 = jnp.sum(f0)

    pr = jnp.clip(jax.nn.sigmoid(hm_ref[0]), 1e-4, 1.0 - 1e-4)
    gt = thm_ref[0]
    pos = (gt == 1.0).astype(jnp.float32)
    neg = (gt < 1.0).astype(jnp.float32)
    onemg = 1.0 - gt
    negw = (onemg * onemg) * (onemg * onemg)
    posl = jnp.sum(jnp.log(pr) * (1.0 - pr) * (1.0 - pr) * pos)
    negl = jnp.sum(jnp.log(1.0 - pr) * pr * pr * negw * neg)
    npos = jnp.sum(pos)

    acc_s[0] = acc_s[0] + s_f0
    acc_s[1] = acc_s[1] + posl
    acc_s[2] = acc_s[2] + negl
    acc_s[3] = acc_s[3] + npos

    cm = cmap_ref[0, 0]
    cmT = cm.T
    prb = jax.nn.sigmoid(cmT)
    posc = ALPHA_ * (1.0 - prb) * (1.0 - prb) * (-jnp.log(prb + 1e-8))
    negc = (1.0 - ALPHA_) * prb * prb * (-jnp.log(1.0 - prb + 1e-8))
    cc = posc - negc

    tof = toff_ref[0]
    ooc = oot_ref[0]
    co = jnp.abs(tof[:, 0, :] - ooc[0])
    co = co + jnp.abs(tof[:, 1, :] - ooc[1])
    co = co + jnp.abs(tof[:, 2, :] - ooc[2])
    co = co + jnp.abs(tof[:, 3, :] - ooc[3])

    col = 0.1 * co + cc
    mn = jnp.min(col)
    ns = lax.broadcasted_iota(jnp.int32, (NTGT, SCHUNK), 0)
    ss = lax.broadcasted_iota(jnp.int32, (NTGT, SCHUNK), 1)
    pm = ss * NTGT + ns
    idx = jnp.min(jnp.where(col == mn, pm, jnp.int32(1 << 30)))
    logit = jnp.sum(jnp.where(pm == idx, cmT, 0.0))

    lane = lax.broadcasted_iota(jnp.int32, (1, 128), 1)
    srcs_scr[0:1, :] = jnp.where(lane == t, idx, srcs_scr[0:1, :])
    logit_scr[0:1, :] = jnp.where(lane == t, logit, logit_scr[0:1, :])
    smatt_scr[pl.ds(t, 1), :] = jnp.full((1, 128), idx, jnp.int32)

    @pl.when(t == NTGT - 1)
    def _finish_batch():
        srow = srcs_scr[0:1, :]
        lrow = logit_scr[0:1, :]
        smat = jnp.broadcast_to(srow, (128, 128))
        smat_t = smatt_scr[...]
        ii = lax.broadcasted_iota(jnp.int32, (128, 128), 0)
        jj = lax.broadcasted_iota(jnp.int32, (128, 128), 1)
        later = ((smat == smat_t) & (ii > jj) & (ii < NTGT)).astype(jnp.int32)
        dead = jnp.max(later, axis=0, keepdims=True)
        keep = (dead == 0) & (lane < NTGT)
        sg = jax.nn.sigmoid(lrow)
        sp_p = _softplus(lrow)
        sp_n = sp_p - lrow
        corr = (ALPHA_ * (1.0 - sg) * (1.0 - sg) * sp_n
                - (1.0 - ALPHA_) * sg * sg * sp_p)
        acc_s[0] = acc_s[0] + jnp.sum(jnp.where(keep, corr, 0.0))
        src_out[0] = jnp.where(lane < NTGT, srow, 0)

    @pl.when((b == BSZ - 1) & (t == NTGT - 1))
    def _emit_acc():
        v = (jnp.where(lane == 0, acc_s[0], 0.0)
             + jnp.where(lane == 1, acc_s[1], 0.0)
             + jnp.where(lane == 2, acc_s[2], 0.0)
             + jnp.where(lane == 3, acc_s[3], 0.0))
        acc_out[...] = v


def _run_tc(tgt_ids, hmrel4, toff, oot, hmrel3, hm3, thm3):
    grid = (BSZ, NTGT)
    return pl.pallas_call(
        _tc_body,
        grid_spec=pltpu.PrefetchScalarGridSpec(
            num_scalar_prefetch=1,
            grid=grid,
            in_specs=[
                pl.BlockSpec((1, 1, HWF // NTGT, NTGT),
                             lambda b, t, ids: (b, ids[b, t], 0, 0)),
                pl.BlockSpec((1, NTGT, 4, SCHUNK),
                             lambda b, t, ids: (b, 0, 0, t)),
                pl.BlockSpec((1, 4, NTGT, SCHUNK),
                             lambda b, t, ids: (b, 0, 0, 0)),
                pl.BlockSpec((1, KCLS, SCHUNK), lambda b, t, ids: (b, 0, t)),
                pl.BlockSpec((1, CHM, SCHUNK), lambda b, t, ids: (b, 0, t)),
                pl.BlockSpec((1, CHM, SCHUNK), lambda b, t, ids: (b, 0, t)),
            ],
            out_specs=[
                pl.BlockSpec((1, 1, 128), lambda b, t, ids: (b, 0, 0)),
                pl.BlockSpec((1, 128), lambda b, t, ids: (0, 0)),
            ],
            scratch_shapes=[
                pltpu.VMEM((8, 128), jnp.int32),
                pltpu.VMEM((8, 128), jnp.float32),
                pltpu.VMEM((128, 128), jnp.int32),
                pltpu.SMEM((8,), jnp.float32),
            ],
        ),
        out_shape=[
            jax.ShapeDtypeStruct((BSZ, 1, 128), jnp.int32),
            jax.ShapeDtypeStruct((1, 128), jnp.float32),
        ],
    )(tgt_ids, hmrel4, toff, oot, hmrel3, hm3, thm3)


def _sc_gather_l1(tab_a, tab_b, idx_a, idx_b, tgt_a, tgt_b, msk):
    n = idx_a.shape[0] * idx_a.shape[1]
    mesh = plsc.VectorSubcoreMesh(core_axis_name="c", subcore_axis_name="s")

    @functools.partial(
        pl.kernel, mesh=mesh,
        out_type=jax.ShapeDtypeStruct((8, 16), jnp.float32),
        scratch_types=[
            pltpu.VMEM(idx_a.shape, jnp.int32),
            pltpu.VMEM(idx_a.shape, jnp.int32),
            pltpu.VMEM((n,), jnp.float32),
            pltpu.VMEM((n,), jnp.float32),
            pltpu.VMEM((n,), jnp.float32),
            pltpu.VMEM((n,), jnp.float32),
            pltpu.VMEM((n,), jnp.float32),
            pltpu.VMEM((8, 16), jnp.float32),
            pltpu.SemaphoreType.DMA,
        ],
    )
    def sck(ta_hbm, tb_hbm, ia_hbm, ib_hbm, tga_hbm, tgb_hbm, m_hbm, out_hbm,
            ia_v, ib_v, ga_v, gb_v, tga_v, tgb_v, m_v, o_v, sem):
        wid = lax.axis_index("s") * 2 + lax.axis_index("c")

        @pl.when(wid == 0)
        def _():
            pltpu.sync_copy(ia_hbm, ia_v)
            pltpu.sync_copy(ib_hbm, ib_v)
            pltpu.sync_copy(tga_hbm, tga_v)
            pltpu.sync_copy(tgb_hbm, tgb_v)
            pltpu.sync_copy(m_hbm, m_v)
            nrows = idx_a.shape[0]
            for r in range(nrows):
                pltpu.async_copy(ta_hbm.at[ia_v.at[r]],
                                 ga_v.at[pl.ds(r * 128, 128)], sem).wait()
                pltpu.async_copy(tb_hbm.at[ib_v.at[r]],
                                 gb_v.at[pl.ds(r * 128, 128)], sem).wait()
            za = jnp.zeros((16,), jnp.float32)
            zb = jnp.zeros((16,), jnp.float32)
            zm = jnp.zeros((16,), jnp.float32)
            zd = jnp.zeros((16,), jnp.float32)
            for i in range(n // 16):
                sl = pl.ds(i * 16, 16)
                mv = m_v[sl]
                ga = ga_v[sl]
                gb = gb_v[sl]
                za = za + jnp.abs(ga * mv - tga_v[sl] * mv)
                zb = zb + jnp.abs(gb * mv - tgb_v[sl] * mv)
                zm = zm + mv
                zd = zd + jnp.abs(ga - gb)
            o_v[0] = za
            o_v[1] = zb
            o_v[2] = zm
            o_v[3] = zd
            zz = jnp.zeros((16,), jnp.float32)
            for r in range(4, 8):
                o_v[r] = zz
            pltpu.sync_copy(o_v, out_hbm)

    return sck(tab_a, tab_b, idx_a, idx_b, tgt_a, tgt_b, msk)


def kernel(out_hm_rel, out_sub_offset, out_obj_offset, out_hm, out_wh,
           out_reg, tgt_hm_rel, tgt_offset, tgt_offset_mask, tgt_hm,
           tgt_reg_mask, tgt_ind, tgt_wh, tgt_reg):
    f32 = jnp.float32
    tgt_ids = tgt_hm_rel.astype(jnp.int32)
    hmrel4 = out_hm_rel.reshape(BSZ, KCLS, SCHUNK, NTGT)
    hmrel3 = out_hm_rel.reshape(BSZ, KCLS, HWF)
    toff = tgt_offset.reshape(BSZ, NTGT, 4, HWF)
    oo4 = jnp.concatenate([out_sub_offset, out_obj_offset], axis=1)
    oot = oo4.reshape(BSZ, 4, SCHUNK, NTGT).transpose(0, 1, 3, 2)
    hm3 = out_hm.reshape(BSZ, CHM, HWF)
    thm3 = tgt_hm.reshape(BSZ, CHM, HWF)

    ind = tgt_ind.astype(jnp.int32)
    bb = jnp.arange(BSZ, dtype=jnp.int32)[:, None, None]
    cc2 = jnp.arange(2, dtype=jnp.int32)[None, None, :]
    idx_wr = ((bb * 2 + cc2) * HWF + ind[:, :, None]).reshape(4, 128)
    tw = tgt_wh.astype(f32).reshape(-1)
    tr = tgt_reg.astype(f32).reshape(-1)
    mexp = jnp.broadcast_to(tgt_reg_mask[:, :, None].astype(f32),
                            (BSZ, MAXO, 2)).reshape(-1)
    sums1 = _sc_gather_l1(out_wh.reshape(-1), out_reg.reshape(-1),
                          idx_wr, idx_wr, tw, tr, mexp)

    src_arr, acc = _run_tc(tgt_ids, hmrel4, toff, oot, hmrel3, hm3, thm3)
    f0sum = acc[0, 0]
    posl = acc[0, 1]
    negl = acc[0, 2]
    npos = acc[0, 3]
    hm_rel_loss = f0sum
    hm_loss = jnp.where(npos > 0, -(posl + negl) / jnp.maximum(npos, 1.0),
                        -negl)

    src2 = src_arr[:, 0, :NTGT]
    mm = jnp.arange(NTGT, dtype=jnp.int32)[None, :, None]
    bb2 = jnp.arange(BSZ, dtype=jnp.int32)[:, None, None]
    cc4 = jnp.arange(4, dtype=jnp.int32)[None, None, :]
    idx_t = (((bb2 * NTGT + mm) * 4 + cc4) * HWF
             + src2[:, :, None]).reshape(4, 128)
    idx_o = ((bb2 * 4 + cc4) * HWF + src2[:, :, None]).reshape(4, 128)
    ones = jnp.ones((BSZ * NTGT * 4,), f32)
    zeros = jnp.zeros((BSZ * NTGT * 4,), f32)

    wh_sum = jnp.sum(sums1[0])
    reg_sum = jnp.sum(sums1[1])
    msum = jnp.sum(sums1[2])
    wh_loss = wh_sum / (msum + 1e-4)
    off_loss = reg_sum / (msum + 1e-4)

    sums_off = _sc_gather_l1(oo4.reshape(-1), tgt_offset.reshape(-1),
                             idx_o, idx_t, zeros, zeros, ones)
    offset_loss = jnp.sum(sums_off[3]) / float(BSZ * NTGT)

    loss = (1.0 * (hm_loss + hm_rel_loss)
            + 0.1 * (wh_loss + offset_loss)
            + 1.0 * off_loss)
    return (loss, hm_loss, wh_loss, off_loss, hm_rel_loss, offset_loss)


# speedup vs baseline: 1.6185x; 1.0224x over previous
import functools

import jax
import jax.numpy as jnp
from jax import lax
from jax.experimental import pallas as pl
from jax.experimental.pallas import tpu as pltpu
from jax.experimental.pallas import tpu_sc as plsc

BSZ, KCLS, HH, WW = 2, 117, 128, 128
HWF = HH * WW
NTGT = 64
MAXO = 128
CHM = 80
ALPHA_ = 0.25
SCHUNK = HWF // NTGT


def _softplus(x):
    return jnp.maximum(x, 0.0) + jnp.log1p(jnp.exp(-jnp.abs(x)))


def _tc_body(ids_ref, cmap_ref, toff_ref,
             sub_ref, obj_ref, hr_ref, hm_ref, thm_ref,
             src_out, acc_out, srcs_scr, logit_scr, smatt_scr, oot_scr,
             acc_v):
    b = pl.program_id(0)
    t = pl.program_id(1)

    @pl.when((b == 0) & (t == 0))
    def _init_acc():
        acc_v[...] = jnp.zeros((8, SCHUNK), jnp.float32)

    @pl.when(t == 0)
    def _init_rows():
        srcs_scr[0:1, :] = jnp.zeros((1, 128), jnp.int32)
        logit_scr[0:1, :] = jnp.zeros((1, 128), jnp.float32)
        oot_scr[0] = sub_ref[0, 0].T
        oot_scr[1] = sub_ref[0, 1].T
        oot_scr[2] = obj_ref[0, 0].T
        oot_scr[3] = obj_ref[0, 1].T

    x = hr_ref[0]
    px = jax.nn.sigmoid(x)
    f0 = (1.0 - ALPHA_) * px * px * _softplus(x)

    pr = jnp.clip(jax.nn.sigmoid(hm_ref[0]), 1e-4, 1.0 - 1e-4)
    gt = thm_ref[0]
    pos = (gt == 1.0).astype(jnp.float32)
    neg = (gt < 1.0).astype(jnp.float32)
    onemg = 1.0 - gt
    negw = (onemg * onemg) * (onemg * onemg)
    posl = jnp.log(pr) * (1.0 - pr) * (1.0 - pr) * pos
    negl = jnp.log(1.0 - pr) * pr * pr * negw * neg

    acc_v[0:1, :] = (acc_v[0:1, :] + jnp.sum(f0, axis=0, keepdims=True))
    acc_v[1:2, :] = (acc_v[1:2, :] + jnp.sum(posl, axis=0, keepdims=True))
    acc_v[2:3, :] = (acc_v[2:3, :] + jnp.sum(negl, axis=0, keepdims=True))
    acc_v[3:4, :] = (acc_v[3:4, :] + jnp.sum(pos, axis=0, keepdims=True))

    cm = cmap_ref[0, 0]
    cmT = cm.T
    prb = jax.nn.sigmoid(cmT)
    posc = ALPHA_ * (1.0 - prb) * (1.0 - prb) * (-jnp.log(prb + 1e-8))
    negc = (1.0 - ALPHA_) * prb * prb * (-jnp.log(1.0 - prb + 1e-8))
    cc = posc - negc

    tof = toff_ref[0]
    co = jnp.abs(tof[:, 0, :] - oot_scr[0])
    co = co + jnp.abs(tof[:, 1, :] - oot_scr[1])
    co = co + jnp.abs(tof[:, 2, :] - oot_scr[2])
    co = co + jnp.abs(tof[:, 3, :] - oot_scr[3])

    col = 0.1 * co + cc
    mn = jnp.min(col)
    ns = lax.broadcasted_iota(jnp.int32, (NTGT, SCHUNK), 0)
    ss = lax.broadcasted_iota(jnp.int32, (NTGT, SCHUNK), 1)
    pm = ss * NTGT + ns
    idx = jnp.min(jnp.where(col == mn, pm, jnp.int32(1 << 30)))
    logit = jnp.sum(jnp.where(pm == idx, cmT, 0.0))

    lane = lax.broadcasted_iota(jnp.int32, (1, 128), 1)
    srcs_scr[0:1, :] = jnp.where(lane == t, idx, srcs_scr[0:1, :])
    logit_scr[0:1, :] = jnp.where(lane == t, logit, logit_scr[0:1, :])
    smatt_scr[pl.ds(t, 1), :] = jnp.full((1, 128), idx, jnp.int32)

    @pl.when(t == NTGT - 1)
    def _finish_batch():
        srow = srcs_scr[0:1, :]
        lrow = logit_scr[0:1, :]
        smat = jnp.broadcast_to(srow, (128, 128))
        smat_t = smatt_scr[...]
        ii = lax.broadcasted_iota(jnp.int32, (128, 128), 0)
        jj = lax.broadcasted_iota(jnp.int32, (128, 128), 1)
        later = ((smat == smat_t) & (ii > jj) & (ii < NTGT)).astype(jnp.int32)
        dead = jnp.max(later, axis=0, keepdims=True)
        keep = (dead == 0) & (lane < NTGT)
        sg = jax.nn.sigmoid(lrow)
        sp_p = _softplus(lrow)
        sp_n = sp_p - lrow
        corr = (ALPHA_ * (1.0 - sg) * (1.0 - sg) * sp_n
                - (1.0 - ALPHA_) * sg * sg * sp_p)
        acc_v[4:5, 0:128] = (acc_v[4:5, 0:128]
                             + jnp.where(keep, corr, 0.0))
        src_out[0] = jnp.where(lane < NTGT, srow, 0)

    @pl.when((b == BSZ - 1) & (t == NTGT - 1))
    def _emit_acc():
        s0 = jnp.sum(acc_v[0:1, :]) + jnp.sum(acc_v[4:5, :])
        s1 = jnp.sum(acc_v[1:2, :])
        s2 = jnp.sum(acc_v[2:3, :])
        s3 = jnp.sum(acc_v[3:4, :])
        v = (jnp.where(lane == 0, s0, 0.0)
             + jnp.where(lane == 1, s1, 0.0)
             + jnp.where(lane == 2, s2, 0.0)
             + jnp.where(lane == 3, s3, 0.0))
        acc_out[...] = v


def _run_tc(tgt_ids, hmrel4, toff, sub4, obj4, hmrel3, hm3, thm3):
    grid = (BSZ, NTGT)

    return pl.pallas_call(
        _tc_body,
        grid_spec=pltpu.PrefetchScalarGridSpec(
            num_scalar_prefetch=1,
            grid=grid,
            in_specs=[
                pl.BlockSpec((1, 1, SCHUNK, NTGT),
                             lambda b, t, ids: (b, ids[b, t], 0, 0)),
                pl.BlockSpec((1, NTGT, 4, SCHUNK),
                             lambda b, t, ids: (b, 0, 0, t)),
                pl.BlockSpec((1, 2, SCHUNK, NTGT),
                             lambda b, t, ids: (b, 0, 0, 0)),
                pl.BlockSpec((1, 2, SCHUNK, NTGT),
                             lambda b, t, ids: (b, 0, 0, 0)),
                pl.BlockSpec((1, KCLS, SCHUNK), lambda b, t, ids: (b, 0, t)),
                pl.BlockSpec((1, CHM, SCHUNK), lambda b, t, ids: (b, 0, t)),
                pl.BlockSpec((1, CHM, SCHUNK), lambda b, t, ids: (b, 0, t)),
            ],
            out_specs=[
                pl.BlockSpec((1, 1, 128), lambda b, t, ids: (b, 0, 0)),
                pl.BlockSpec((1, 128), lambda b, t, ids: (0, 0)),
            ],
            scratch_shapes=[
                pltpu.VMEM((8, 128), jnp.int32),
                pltpu.VMEM((8, 128), jnp.float32),
                pltpu.VMEM((128, 128), jnp.int32),
                pltpu.VMEM((4, NTGT, SCHUNK), jnp.float32),
                pltpu.VMEM((8, SCHUNK), jnp.float32),
            ],
        ),
        out_shape=[
            jax.ShapeDtypeStruct((BSZ, 1, 128), jnp.int32),
            jax.ShapeDtypeStruct((1, 128), jnp.float32),
        ],
    )(tgt_ids, hmrel4, toff, sub4, obj4, hmrel3, hm3, thm3)


def _sc_gather_l1(tab_a, tab_b, idx_a, idx_b, tgt_a, tgt_b, msk):
    n = idx_a.shape[0] * idx_a.shape[1]
    mesh = plsc.VectorSubcoreMesh(core_axis_name="c", subcore_axis_name="s")

    @functools.partial(
        pl.kernel, mesh=mesh,
        out_type=jax.ShapeDtypeStruct((8, 16), jnp.float32),
        scratch_types=[
            pltpu.VMEM(idx_a.shape, jnp.int32),
            pltpu.VMEM(idx_a.shape, jnp.int32),
            pltpu.VMEM((n,), jnp.float32),
            pltpu.VMEM((n,), jnp.float32),
            pltpu.VMEM((n,), jnp.float32),
            pltpu.VMEM((n,), jnp.float32),
            pltpu.VMEM((n,), jnp.float32),
            pltpu.VMEM((8, 16), jnp.float32),
            pltpu.SemaphoreType.DMA,
        ],
    )
    def sck(ta_hbm, tb_hbm, ia_hbm, ib_hbm, tga_hbm, tgb_hbm, m_hbm, out_hbm,
            ia_v, ib_v, ga_v, gb_v, tga_v, tgb_v, m_v, o_v, sem):
        wid = lax.axis_index("s") * 2 + lax.axis_index("c")

        @pl.when(wid == 0)
        def _():
            pltpu.sync_copy(ia_hbm, ia_v)
            pltpu.sync_copy(ib_hbm, ib_v)
            pltpu.sync_copy(tga_hbm, tga_v)
            pltpu.sync_copy(tgb_hbm, tgb_v)
            pltpu.sync_copy(m_hbm, m_v)
            nrows = idx_a.shape[0]
            for r in range(nrows):
                pltpu.async_copy(ta_hbm.at[ia_v.at[r]],
                                 ga_v.at[pl.ds(r * 128, 128)], sem).wait()
                pltpu.async_copy(tb_hbm.at[ib_v.at[r]],
                                 gb_v.at[pl.ds(r * 128, 128)], sem).wait()
            za = jnp.zeros((16,), jnp.float32)
            zb = jnp.zeros((16,), jnp.float32)
            zm = jnp.zeros((16,), jnp.float32)
            zd = jnp.zeros((16,), jnp.float32)
            for i in range(n // 16):
                sl = pl.ds(i * 16, 16)
                mv = m_v[sl]
                ga = ga_v[sl]
                gb = gb_v[sl]
                za = za + jnp.abs(ga * mv - tga_v[sl] * mv)
                zb = zb + jnp.abs(gb * mv - tgb_v[sl] * mv)
                zm = zm + mv
                zd = zd + jnp.abs(ga - gb)
            o_v[0] = za
            o_v[1] = zb
            o_v[2] = zm
            o_v[3] = zd
            zz = jnp.zeros((16,), jnp.float32)
            for r in range(4, 8):
                o_v[r] = zz
            pltpu.sync_copy(o_v, out_hbm)

    return sck(tab_a, tab_b, idx_a, idx_b, tgt_a, tgt_b, msk)


def _sc_offset_l1(sub_t, obj_t, tgt_t, idx_so, idx_t1, idx_t2):
    n = idx_so.shape[0] * idx_so.shape[1]
    mesh = plsc.VectorSubcoreMesh(core_axis_name="c", subcore_axis_name="s")

    @functools.partial(
        pl.kernel, mesh=mesh,
        out_type=jax.ShapeDtypeStruct((8, 16), jnp.float32),
        scratch_types=[
            pltpu.VMEM(idx_so.shape, jnp.int32),
            pltpu.VMEM(idx_so.shape, jnp.int32),
            pltpu.VMEM(idx_so.shape, jnp.int32),
            pltpu.VMEM((n,), jnp.float32),
            pltpu.VMEM((n,), jnp.float32),
            pltpu.VMEM((n,), jnp.float32),
            pltpu.VMEM((n,), jnp.float32),
            pltpu.VMEM((8, 16), jnp.float32),
            pltpu.SemaphoreType.DMA,
        ],
    )
    def sck(sub_hbm, obj_hbm, tgt_hbm, iso_hbm, it1_hbm, it2_hbm, out_hbm,
            iso_v, it1_v, it2_v, gs_v, go_v, g1_v, g2_v, o_v, sem):
        wid = lax.axis_index("s") * 2 + lax.axis_index("c")

        @pl.when(wid == 0)
        def _():
            pltpu.sync_copy(iso_hbm, iso_v)
            pltpu.sync_copy(it1_hbm, it1_v)
            pltpu.sync_copy(it2_hbm, it2_v)
            for r in range(idx_so.shape[0]):
                dst = pl.ds(r * 128, 128)
                pltpu.async_copy(sub_hbm.at[iso_v.at[r]], gs_v.at[dst],
                                 sem).wait()
                pltpu.async_copy(obj_hbm.at[iso_v.at[r]], go_v.at[dst],
                                 sem).wait()
                pltpu.async_copy(tgt_hbm.at[it1_v.at[r]], g1_v.at[dst],
                                 sem).wait()
                pltpu.async_copy(tgt_hbm.at[it2_v.at[r]], g2_v.at[dst],
                                 sem).wait()
            zz = jnp.zeros((16,), jnp.float32)
            zd = zz
            for i in range(n // 16):
                sl = pl.ds(i * 16, 16)
                zd = (zd + jnp.abs(gs_v[sl] - g1_v[sl])
                      + jnp.abs(go_v[sl] - g2_v[sl]))
            o_v[0] = zd
            for r in range(1, 8):
                o_v[r] = zz
            pltpu.sync_copy(o_v, out_hbm)

    return sck(sub_t, obj_t, tgt_t, idx_so, idx_t1, idx_t2)


def kernel(out_hm_rel, out_sub_offset, out_obj_offset, out_hm, out_wh,
           out_reg, tgt_hm_rel, tgt_offset, tgt_offset_mask, tgt_hm,
           tgt_reg_mask, tgt_ind, tgt_wh, tgt_reg):
    f32 = jnp.float32
    tgt_ids = tgt_hm_rel.astype(jnp.int32)
    hmrel4 = out_hm_rel.reshape(BSZ, KCLS, SCHUNK, NTGT)
    hmrel3 = out_hm_rel.reshape(BSZ, KCLS, HWF)
    toff = tgt_offset.reshape(BSZ, NTGT, 4, HWF)
    sub4 = out_sub_offset.reshape(BSZ, 2, SCHUNK, NTGT)
    obj4 = out_obj_offset.reshape(BSZ, 2, SCHUNK, NTGT)
    hm3 = out_hm.reshape(BSZ, CHM, HWF)
    thm3 = tgt_hm.reshape(BSZ, CHM, HWF)

    ind = tgt_ind.astype(jnp.int32)
    bb = jnp.arange(BSZ, dtype=jnp.int32)[:, None, None]
    cc2 = jnp.arange(2, dtype=jnp.int32)[None, None, :]
    idx_wr = ((bb * 2 + cc2) * HWF + ind[:, :, None]).reshape(4, 128)
    tw = tgt_wh.astype(f32).reshape(-1)
    tr = tgt_reg.astype(f32).reshape(-1)
    mexp = jnp.broadcast_to(tgt_reg_mask[:, :, None].astype(f32),
                            (BSZ, MAXO, 2)).reshape(-1)
    sums1 = _sc_gather_l1(out_wh.reshape(-1), out_reg.reshape(-1),
                          idx_wr, idx_wr, tw, tr, mexp)

    src_arr, acc = _run_tc(tgt_ids, hmrel4, toff, sub4, obj4, hmrel3, hm3,
                           thm3)
    f0sum = acc[0, 0]
    posl = acc[0, 1]
    negl = acc[0, 2]
    npos = acc[0, 3]
    hm_rel_loss = f0sum
    hm_loss = jnp.where(npos > 0, -(posl + negl) / jnp.maximum(npos, 1.0),
                        -negl)

    src2 = src_arr[:, 0, :NTGT]
    mm = jnp.arange(NTGT, dtype=jnp.int32)[None, :, None]
    bb2 = jnp.arange(BSZ, dtype=jnp.int32)[:, None, None]
    idx_so = ((bb2 * 2 + cc2) * HWF + src2[:, :, None]).reshape(2, 128)
    idx_t1 = (((bb2 * NTGT + mm) * 4 + cc2) * HWF
              + src2[:, :, None]).reshape(2, 128)
    idx_t2 = (((bb2 * NTGT + mm) * 4 + cc2 + 2) * HWF
              + src2[:, :, None]).reshape(2, 128)

    wh_sum = jnp.sum(sums1[0])
    reg_sum = jnp.sum(sums1[1])
    msum = jnp.sum(sums1[2])
    wh_loss = wh_sum / (msum + 1e-4)
    off_loss = reg_sum / (msum + 1e-4)

    sums_off = _sc_offset_l1(out_sub_offset.reshape(-1),
                             out_obj_offset.reshape(-1),
                             tgt_offset.reshape(-1),
                             idx_so, idx_t1, idx_t2)
    offset_loss = jnp.sum(sums_off[0]) / float(BSZ * NTGT)

    loss = (1.0 * (hm_loss + hm_rel_loss)
            + 0.1 * (wh_loss + offset_loss)
            + 1.0 * off_loss)
    return (loss, hm_loss, wh_loss, off_loss, hm_rel_loss, offset_loss)


# speedup vs baseline: 1.6754x; 1.0352x over previous
import functools

import jax
import jax.numpy as jnp
from jax import lax
from jax.experimental import pallas as pl
from jax.experimental.pallas import tpu as pltpu
from jax.experimental.pallas import tpu_sc as plsc

BSZ, KCLS, HH, WW = 2, 117, 128, 128
HWF = HH * WW
NTGT = 64
MAXO = 128
CHM = 80
ALPHA_ = 0.25
SCHUNK = HWF // NTGT


def _softplus(x):
    return jnp.maximum(x, 0.0) + jnp.log1p(jnp.exp(-jnp.abs(x)))


def _tc_body(ids_ref, cmap_ref, toff_ref,
             sub_ref, obj_ref, hr_ref, hm_ref, thm_ref,
             src_out, acc_out, srcs_scr, logit_scr, smatt_scr, oot_scr,
             acc_v):
    b = pl.program_id(0)
    t = pl.program_id(1)

    @pl.when((b == 0) & (t == 0))
    def _init_acc():
        acc_v[...] = jnp.zeros((8, SCHUNK), jnp.float32)

    @pl.when(t == 0)
    def _init_rows():
        srcs_scr[0:1, :] = jnp.zeros((1, 128), jnp.int32)
        logit_scr[0:1, :] = jnp.zeros((1, 128), jnp.float32)
        oot_scr[:, 0, :] = sub_ref[0, 0].T
        oot_scr[:, 1, :] = sub_ref[0, 1].T
        oot_scr[:, 2, :] = obj_ref[0, 0].T
        oot_scr[:, 3, :] = obj_ref[0, 1].T

    x = hr_ref[0]
    ex = jnp.exp(-jnp.abs(x))
    rx = 1.0 / (1.0 + ex)
    px = jnp.where(x >= 0, rx, 1.0 - rx)
    f0 = ((1.0 - ALPHA_) * px * px
          * (jnp.maximum(x, 0.0) + jnp.log1p(ex)))

    y = hm_ref[0]
    ey = jnp.exp(-jnp.abs(y))
    ry = 1.0 / (1.0 + ey)
    sy = jnp.where(y >= 0, ry, 1.0 - ry)
    pr = jnp.clip(sy, 1e-4, 1.0 - 1e-4)
    spy = jnp.log1p(ey)
    lcl = -9.210340371976182
    lch = -1.0000500033334732e-04
    logp = jnp.clip(jnp.minimum(y, 0.0) - spy, lcl, lch)
    log1mp = jnp.clip(-jnp.maximum(y, 0.0) - spy, lcl, lch)
    gt = thm_ref[0]
    pos = (gt == 1.0).astype(jnp.float32)
    neg = (gt < 1.0).astype(jnp.float32)
    onemg = 1.0 - gt
    negw = (onemg * onemg) * (onemg * onemg)
    posl = logp * (1.0 - pr) * (1.0 - pr) * pos
    negl = log1mp * pr * pr * negw * neg

    acc_v[0:1, :] = (acc_v[0:1, :] + jnp.sum(f0, axis=0, keepdims=True))
    acc_v[1:2, :] = (acc_v[1:2, :] + jnp.sum(posl, axis=0, keepdims=True))
    acc_v[2:3, :] = (acc_v[2:3, :] + jnp.sum(negl, axis=0, keepdims=True))
    acc_v[3:4, :] = (acc_v[3:4, :] + jnp.sum(pos, axis=0, keepdims=True))

    cm = cmap_ref[0, 0]
    cmT = cm.T
    ec = jnp.exp(-jnp.abs(cmT))
    rc = 1.0 / (1.0 + ec)
    prb = jnp.where(cmT >= 0, rc, 1.0 - rc)
    spc = jnp.log1p(ec)
    posc = ALPHA_ * (1.0 - prb) * (1.0 - prb) * (jnp.maximum(-cmT, 0.0) + spc)
    negc = (1.0 - ALPHA_) * prb * prb * (jnp.maximum(cmT, 0.0) + spc)
    cc = posc - negc

    diff = jnp.abs(toff_ref[0] - oot_scr[...])
    diff2 = diff.reshape(4 * NTGT, SCHUNK)
    rr = lax.broadcasted_iota(jnp.int32, (NTGT, 4 * NTGT), 1)
    nn2 = lax.broadcasted_iota(jnp.int32, (NTGT, 4 * NTGT), 0)
    sel = ((rr >> 2) == nn2).astype(jnp.float32)
    co = lax.dot_general(sel, diff2, (((1,), (0,)), ((), ())),
                         preferred_element_type=jnp.float32)

    col = 0.1 * co + cc
    mn = jnp.min(col)
    ns = lax.broadcasted_iota(jnp.int32, (NTGT, SCHUNK), 0)
    ss = lax.broadcasted_iota(jnp.int32, (NTGT, SCHUNK), 1)
    pm = ss * NTGT + ns
    idx = jnp.min(jnp.where(col == mn, pm, jnp.int32(1 << 30)))
    logit = jnp.sum(jnp.where(pm == idx, cmT, 0.0))

    lane = lax.broadcasted_iota(jnp.int32, (1, 128), 1)
    srcs_scr[0:1, :] = jnp.where(lane == t, idx, srcs_scr[0:1, :])
    logit_scr[0:1, :] = jnp.where(lane == t, logit, logit_scr[0:1, :])
    smatt_scr[pl.ds(t, 1), :] = jnp.full((1, 128), idx, jnp.int32)

    @pl.when(t == NTGT - 1)
    def _finish_batch():
        srow = srcs_scr[0:1, :]
        lrow = logit_scr[0:1, :]
        smat = jnp.broadcast_to(srow, (128, 128))
        smat_t = smatt_scr[...]
        ii = lax.broadcasted_iota(jnp.int32, (128, 128), 0)
        jj = lax.broadcasted_iota(jnp.int32, (128, 128), 1)
        later = ((smat == smat_t) & (ii > jj) & (ii < NTGT)).astype(jnp.int32)
        dead = jnp.max(later, axis=0, keepdims=True)
        keep = (dead == 0) & (lane < NTGT)
        sg = jax.nn.sigmoid(lrow)
        sp_p = _softplus(lrow)
        sp_n = sp_p - lrow
        corr = (ALPHA_ * (1.0 - sg) * (1.0 - sg) * sp_n
                - (1.0 - ALPHA_) * sg * sg * sp_p)
        acc_v[4:5, 0:128] = (acc_v[4:5, 0:128]
                             + jnp.where(keep, corr, 0.0))
        src_out[0] = jnp.where(lane < NTGT, srow, 0)

    @pl.when((b == BSZ - 1) & (t == NTGT - 1))
    def _emit_acc():
        s0 = jnp.sum(acc_v[0:1, :]) + jnp.sum(acc_v[4:5, :])
        s1 = jnp.sum(acc_v[1:2, :])
        s2 = jnp.sum(acc_v[2:3, :])
        s3 = jnp.sum(acc_v[3:4, :])
        v = (jnp.where(lane == 0, s0, 0.0)
             + jnp.where(lane == 1, s1, 0.0)
             + jnp.where(lane == 2, s2, 0.0)
             + jnp.where(lane == 3, s3, 0.0))
        acc_out[...] = v


def _run_tc(tgt_ids, hmrel4, toff, sub4, obj4, hmrel3, hm3, thm3):
    grid = (BSZ, NTGT)

    return pl.pallas_call(
        _tc_body,
        grid_spec=pltpu.PrefetchScalarGridSpec(
            num_scalar_prefetch=1,
            grid=grid,
            in_specs=[
                pl.BlockSpec((1, 1, SCHUNK, NTGT),
                             lambda b, t, ids: (b, ids[b, t], 0, 0)),
                pl.BlockSpec((1, NTGT, 4, SCHUNK),
                             lambda b, t, ids: (b, 0, 0, t)),
                pl.BlockSpec((1, 2, SCHUNK, NTGT),
                             lambda b, t, ids: (b, 0, 0, 0)),
                pl.BlockSpec((1, 2, SCHUNK, NTGT),
                             lambda b, t, ids: (b, 0, 0, 0)),
                pl.BlockSpec((1, KCLS, SCHUNK), lambda b, t, ids: (b, 0, t)),
                pl.BlockSpec((1, CHM, SCHUNK), lambda b, t, ids: (b, 0, t)),
                pl.BlockSpec((1, CHM, SCHUNK), lambda b, t, ids: (b, 0, t)),
            ],
            out_specs=[
                pl.BlockSpec((1, 1, 128), lambda b, t, ids: (b, 0, 0)),
                pl.BlockSpec((1, 128), lambda b, t, ids: (0, 0)),
            ],
            scratch_shapes=[
                pltpu.VMEM((8, 128), jnp.int32),
                pltpu.VMEM((8, 128), jnp.float32),
                pltpu.VMEM((128, 128), jnp.int32),
                pltpu.VMEM((NTGT, 4, SCHUNK), jnp.float32),
                pltpu.VMEM((8, SCHUNK), jnp.float32),
            ],
        ),
        out_shape=[
            jax.ShapeDtypeStruct((BSZ, 1, 128), jnp.int32),
            jax.ShapeDtypeStruct((1, 128), jnp.float32),
        ],
    )(tgt_ids, hmrel4, toff, sub4, obj4, hmrel3, hm3, thm3)


def _sc_gather_l1(tab_a, tab_b, idx_a, idx_b, tgt_a, tgt_b, msk):
    n = idx_a.shape[0] * idx_a.shape[1]
    mesh = plsc.VectorSubcoreMesh(core_axis_name="c", subcore_axis_name="s")

    @functools.partial(
        pl.kernel, mesh=mesh,
        out_type=jax.ShapeDtypeStruct((8, 16), jnp.float32),
        scratch_types=[
            pltpu.VMEM(idx_a.shape, jnp.int32),
            pltpu.VMEM(idx_a.shape, jnp.int32),
            pltpu.VMEM((n,), jnp.float32),
            pltpu.VMEM((n,), jnp.float32),
            pltpu.VMEM((n,), jnp.float32),
            pltpu.VMEM((n,), jnp.float32),
            pltpu.VMEM((n,), jnp.float32),
            pltpu.VMEM((8, 16), jnp.float32),
            pltpu.SemaphoreType.DMA,
        ],
    )
    def sck(ta_hbm, tb_hbm, ia_hbm, ib_hbm, tga_hbm, tgb_hbm, m_hbm, out_hbm,
            ia_v, ib_v, ga_v, gb_v, tga_v, tgb_v, m_v, o_v, sem):
        wid = lax.axis_index("s") * 2 + lax.axis_index("c")

        @pl.when(wid == 0)
        def _():
            pltpu.sync_copy(ia_hbm, ia_v)
            pltpu.sync_copy(ib_hbm, ib_v)
            pltpu.sync_copy(tga_hbm, tga_v)
            pltpu.sync_copy(tgb_hbm, tgb_v)
            pltpu.sync_copy(m_hbm, m_v)
            nrows = idx_a.shape[0]
            for r in range(nrows):
                pltpu.async_copy(ta_hbm.at[ia_v.at[r]],
                                 ga_v.at[pl.ds(r * 128, 128)], sem).wait()
                pltpu.async_copy(tb_hbm.at[ib_v.at[r]],
                                 gb_v.at[pl.ds(r * 128, 128)], sem).wait()
            za = jnp.zeros((16,), jnp.float32)
            zb = jnp.zeros((16,), jnp.float32)
            zm = jnp.zeros((16,), jnp.float32)
            zd = jnp.zeros((16,), jnp.float32)
            for i in range(n // 16):
                sl = pl.ds(i * 16, 16)
                mv = m_v[sl]
                ga = ga_v[sl]
                gb = gb_v[sl]
                za = za + jnp.abs(ga * mv - tga_v[sl] * mv)
                zb = zb + jnp.abs(gb * mv - tgb_v[sl] * mv)
                zm = zm + mv
                zd = zd + jnp.abs(ga - gb)
            o_v[0] = za
            o_v[1] = zb
            o_v[2] = zm
            o_v[3] = zd
            zz = jnp.zeros((16,), jnp.float32)
            for r in range(4, 8):
                o_v[r] = zz
            pltpu.sync_copy(o_v, out_hbm)

    return sck(tab_a, tab_b, idx_a, idx_b, tgt_a, tgt_b, msk)


def _sc_offset_l1(sub_t, obj_t, tgt_t, idx_so, idx_t1, idx_t2):
    n = idx_so.shape[0] * idx_so.shape[1]
    mesh = plsc.VectorSubcoreMesh(core_axis_name="c", subcore_axis_name="s")

    @functools.partial(
        pl.kernel, mesh=mesh,
        out_type=jax.ShapeDtypeStruct((8, 16), jnp.float32),
        scratch_types=[
            pltpu.VMEM(idx_so.shape, jnp.int32),
            pltpu.VMEM(idx_so.shape, jnp.int32),
            pltpu.VMEM(idx_so.shape, jnp.int32),
            pltpu.VMEM((n,), jnp.float32),
            pltpu.VMEM((n,), jnp.float32),
            pltpu.VMEM((n,), jnp.float32),
            pltpu.VMEM((n,), jnp.float32),
            pltpu.VMEM((8, 16), jnp.float32),
            pltpu.SemaphoreType.DMA,
        ],
    )
    def sck(sub_hbm, obj_hbm, tgt_hbm, iso_hbm, it1_hbm, it2_hbm, out_hbm,
            iso_v, it1_v, it2_v, gs_v, go_v, g1_v, g2_v, o_v, sem):
        wid = lax.axis_index("s") * 2 + lax.axis_index("c")

        @pl.when(wid == 0)
        def _():
            pltpu.sync_copy(iso_hbm, iso_v)
            pltpu.sync_copy(it1_hbm, it1_v)
            pltpu.sync_copy(it2_hbm, it2_v)
            for r in range(idx_so.shape[0]):
                dst = pl.ds(r * 128, 128)
                pltpu.async_copy(sub_hbm.at[iso_v.at[r]], gs_v.at[dst],
                                 sem).wait()
                pltpu.async_copy(obj_hbm.at[iso_v.at[r]], go_v.at[dst],
                                 sem).wait()
                pltpu.async_copy(tgt_hbm.at[it1_v.at[r]], g1_v.at[dst],
                                 sem).wait()
                pltpu.async_copy(tgt_hbm.at[it2_v.at[r]], g2_v.at[dst],
                                 sem).wait()
            zz = jnp.zeros((16,), jnp.float32)
            zd = zz
            for i in range(n // 16):
                sl = pl.ds(i * 16, 16)
                zd = (zd + jnp.abs(gs_v[sl] - g1_v[sl])
                      + jnp.abs(go_v[sl] - g2_v[sl]))
            o_v[0] = zd
            for r in range(1, 8):
                o_v[r] = zz
            pltpu.sync_copy(o_v, out_hbm)

    return sck(sub_t, obj_t, tgt_t, idx_so, idx_t1, idx_t2)


def kernel(out_hm_rel, out_sub_offset, out_obj_offset, out_hm, out_wh,
           out_reg, tgt_hm_rel, tgt_offset, tgt_offset_mask, tgt_hm,
           tgt_reg_mask, tgt_ind, tgt_wh, tgt_reg):
    f32 = jnp.float32
    tgt_ids = tgt_hm_rel.astype(jnp.int32)
    hmrel4 = out_hm_rel.reshape(BSZ, KCLS, SCHUNK, NTGT)
    hmrel3 = out_hm_rel.reshape(BSZ, KCLS, HWF)
    toff = tgt_offset.reshape(BSZ, NTGT, 4, HWF)
    sub4 = out_sub_offset.reshape(BSZ, 2, SCHUNK, NTGT)
    obj4 = out_obj_offset.reshape(BSZ, 2, SCHUNK, NTGT)
    hm3 = out_hm.reshape(BSZ, CHM, HWF)
    thm3 = tgt_hm.reshape(BSZ, CHM, HWF)

    ind = tgt_ind.astype(jnp.int32)
    bb = jnp.arange(BSZ, dtype=jnp.int32)[:, None, None]
    cc2 = jnp.arange(2, dtype=jnp.int32)[None, None, :]
    idx_wr = ((bb * 2 + cc2) * HWF + ind[:, :, None]).reshape(4, 128)
    tw = tgt_wh.astype(f32).reshape(-1)
    tr = tgt_reg.astype(f32).reshape(-1)
    mexp = jnp.broadcast_to(tgt_reg_mask[:, :, None].astype(f32),
                            (BSZ, MAXO, 2)).reshape(-1)
    sums1 = _sc_gather_l1(out_wh.reshape(-1), out_reg.reshape(-1),
                          idx_wr, idx_wr, tw, tr, mexp)

    src_arr, acc = _run_tc(tgt_ids, hmrel4, toff, sub4, obj4, hmrel3, hm3,
                           thm3)
    f0sum = acc[0, 0]
    posl = acc[0, 1]
    negl = acc[0, 2]
    npos = acc[0, 3]
    hm_rel_loss = f0sum
    hm_loss = jnp.where(npos > 0, -(posl + negl) / jnp.maximum(npos, 1.0),
                        -negl)

    src2 = src_arr[:, 0, :NTGT]
    mm = jnp.arange(NTGT, dtype=jnp.int32)[None, :, None]
    bb2 = jnp.arange(BSZ, dtype=jnp.int32)[:, None, None]
    idx_so = ((bb2 * 2 + cc2) * HWF + src2[:, :, None]).reshape(2, 128)
    idx_t1 = (((bb2 * NTGT + mm) * 4 + cc2) * HWF
              + src2[:, :, None]).reshape(2, 128)
    idx_t2 = (((bb2 * NTGT + mm) * 4 + cc2 + 2) * HWF
              + src2[:, :, None]).reshape(2, 128)

    wh_sum = jnp.sum(sums1[0])
    reg_sum = jnp.sum(sums1[1])
    msum = jnp.sum(sums1[2])
    wh_loss = wh_sum / (msum + 1e-4)
    off_loss = reg_sum / (msum + 1e-4)

    sums_off = _sc_offset_l1(out_sub_offset.reshape(-1),
                             out_obj_offset.reshape(-1),
                             tgt_offset.reshape(-1),
                             idx_so, idx_t1, idx_t2)
    offset_loss = jnp.sum(sums_off[0]) / float(BSZ * NTGT)

    loss = (1.0 * (hm_loss + hm_rel_loss)
            + 0.1 * (wh_loss + offset_loss)
            + 1.0 * off_loss)
    return (loss, hm_loss, wh_loss, off_loss, hm_rel_loss, offset_loss)


# speedup vs baseline: 1.6843x; 1.0053x over previous
import functools

import jax
import jax.numpy as jnp
from jax import lax
from jax.experimental import pallas as pl
from jax.experimental.pallas import tpu as pltpu
from jax.experimental.pallas import tpu_sc as plsc

BSZ, KCLS, HH, WW = 2, 117, 128, 128
HWF = HH * WW
NTGT = 64
MAXO = 128
CHM = 80
ALPHA_ = 0.25
SCHUNK = HWF // NTGT


def _softplus(x):
    return jnp.maximum(x, 0.0) + jnp.log1p(jnp.exp(-jnp.abs(x)))


def _tc_body(ids_ref, cmap_ref, toff_ref,
             sub_ref, obj_ref, hr_ref, hm_ref, thm_ref,
             src_out, acc_out, srcs_scr, logit_scr, smatt_scr, oot_scr,
             acc_v):
    b = pl.program_id(0)
    t = pl.program_id(1)

    @pl.when((b == 0) & (t == 0))
    def _init_acc():
        acc_v[...] = jnp.zeros((8, SCHUNK), jnp.float32)

    @pl.when(t == 0)
    def _init_rows():
        srcs_scr[0:1, :] = jnp.zeros((1, 128), jnp.int32)
        logit_scr[0:1, :] = jnp.zeros((1, 128), jnp.float32)
        oot_scr[:, 0, :] = sub_ref[0, 0].T
        oot_scr[:, 1, :] = sub_ref[0, 1].T
        oot_scr[:, 2, :] = obj_ref[0, 0].T
        oot_scr[:, 3, :] = obj_ref[0, 1].T

    x = hr_ref[0]
    ex = jnp.exp(-jnp.abs(x))
    rx = 1.0 / (1.0 + ex)
    px = jnp.where(x >= 0, rx, 1.0 - rx)
    f0 = ((1.0 - ALPHA_) * px * px
          * (jnp.maximum(x, 0.0) + jnp.log1p(ex)))

    y = hm_ref[0]
    ey = jnp.exp(-jnp.abs(y))
    ry = 1.0 / (1.0 + ey)
    sy = jnp.where(y >= 0, ry, 1.0 - ry)
    pr = jnp.clip(sy, 1e-4, 1.0 - 1e-4)
    spy = jnp.log1p(ey)
    lcl = -9.210340371976182
    lch = -1.0000500033334732e-04
    logp = jnp.clip(jnp.minimum(y, 0.0) - spy, lcl, lch)
    log1mp = jnp.clip(-jnp.maximum(y, 0.0) - spy, lcl, lch)
    gt = thm_ref[0]
    pos = (gt == 1.0).astype(jnp.float32)
    neg = (gt < 1.0).astype(jnp.float32)
    onemg = 1.0 - gt
    negw = (onemg * onemg) * (onemg * onemg)
    posl = logp * (1.0 - pr) * (1.0 - pr) * pos
    negl = log1mp * pr * pr * negw * neg

    acc_v[0:1, :] = (acc_v[0:1, :] + jnp.sum(f0, axis=0, keepdims=True))
    acc_v[1:2, :] = (acc_v[1:2, :] + jnp.sum(posl, axis=0, keepdims=True))
    acc_v[2:3, :] = (acc_v[2:3, :] + jnp.sum(negl, axis=0, keepdims=True))
    acc_v[3:4, :] = (acc_v[3:4, :] + jnp.sum(pos, axis=0, keepdims=True))

    cm = cmap_ref[0, 0]
    cmT = cm.T
    ec = jnp.exp(-jnp.abs(cmT))
    rc = 1.0 / (1.0 + ec)
    prb = jnp.where(cmT >= 0, rc, 1.0 - rc)
    spc = jnp.log1p(ec)
    posc = ALPHA_ * (1.0 - prb) * (1.0 - prb) * (jnp.maximum(-cmT, 0.0) + spc)
    negc = (1.0 - ALPHA_) * prb * prb * (jnp.maximum(cmT, 0.0) + spc)
    cc = posc - negc

    diff = jnp.abs(toff_ref[0] - oot_scr[...])
    diff2 = diff.reshape(4 * NTGT, SCHUNK)
    rr = lax.broadcasted_iota(jnp.int32, (NTGT, 4 * NTGT), 1)
    nn2 = lax.broadcasted_iota(jnp.int32, (NTGT, 4 * NTGT), 0)
    sel = ((rr >> 2) == nn2).astype(jnp.float32)
    co = lax.dot_general(sel, diff2, (((1,), (0,)), ((), ())),
                         preferred_element_type=jnp.float32)

    col = 0.1 * co + cc
    mn = jnp.min(col)
    ns = lax.broadcasted_iota(jnp.int32, (NTGT, SCHUNK), 0)
    ss = lax.broadcasted_iota(jnp.int32, (NTGT, SCHUNK), 1)
    pm = ss * NTGT + ns
    idx = jnp.min(jnp.where(col == mn, pm, jnp.int32(1 << 30)))
    logit = jnp.sum(jnp.where(pm == idx, cmT, 0.0))

    lane = lax.broadcasted_iota(jnp.int32, (1, 128), 1)
    srcs_scr[0:1, :] = jnp.where(lane == t, idx, srcs_scr[0:1, :])
    logit_scr[0:1, :] = jnp.where(lane == t, logit, logit_scr[0:1, :])
    smatt_scr[pl.ds(t, 1), :] = jnp.full((1, 128), idx, jnp.int32)

    @pl.when(t == NTGT - 1)
    def _finish_batch():
        srow = srcs_scr[0:1, :]
        lrow = logit_scr[0:1, :]
        smat = jnp.broadcast_to(srow, (128, 128))
        smat_t = smatt_scr[...]
        ii = lax.broadcasted_iota(jnp.int32, (128, 128), 0)
        jj = lax.broadcasted_iota(jnp.int32, (128, 128), 1)
        later = ((smat == smat_t) & (ii > jj) & (ii < NTGT)).astype(jnp.int32)
        dead = jnp.max(later, axis=0, keepdims=True)
        keep = (dead == 0) & (lane < NTGT)
        sg = jax.nn.sigmoid(lrow)
        sp_p = _softplus(lrow)
        sp_n = sp_p - lrow
        corr = (ALPHA_ * (1.0 - sg) * (1.0 - sg) * sp_n
                - (1.0 - ALPHA_) * sg * sg * sp_p)
        acc_v[4:5, 0:128] = (acc_v[4:5, 0:128]
                             + jnp.where(keep, corr, 0.0))
        src_out[0] = jnp.where(lane < NTGT, srow, 0)

    @pl.when((b == BSZ - 1) & (t == NTGT - 1))
    def _emit_acc():
        s0 = jnp.sum(acc_v[0:1, :]) + jnp.sum(acc_v[4:5, :])
        s1 = jnp.sum(acc_v[1:2, :])
        s2 = jnp.sum(acc_v[2:3, :])
        s3 = jnp.sum(acc_v[3:4, :])
        v = (jnp.where(lane == 0, s0, 0.0)
             + jnp.where(lane == 1, s1, 0.0)
             + jnp.where(lane == 2, s2, 0.0)
             + jnp.where(lane == 3, s3, 0.0))
        acc_out[...] = v


def _run_tc(tgt_ids, hmrel4, toff, sub4, obj4, hmrel3, hm3, thm3):
    grid = (BSZ, NTGT)

    return pl.pallas_call(
        _tc_body,
        grid_spec=pltpu.PrefetchScalarGridSpec(
            num_scalar_prefetch=1,
            grid=grid,
            in_specs=[
                pl.BlockSpec((1, 1, SCHUNK, NTGT),
                             lambda b, t, ids: (b, ids[b, t], 0, 0)),
                pl.BlockSpec((1, NTGT, 4, SCHUNK),
                             lambda b, t, ids: (b, 0, 0, t)),
                pl.BlockSpec((1, 2, SCHUNK, NTGT),
                             lambda b, t, ids: (b, 0, 0, 0)),
                pl.BlockSpec((1, 2, SCHUNK, NTGT),
                             lambda b, t, ids: (b, 0, 0, 0)),
                pl.BlockSpec((1, KCLS, SCHUNK), lambda b, t, ids: (b, 0, t)),
                pl.BlockSpec((1, CHM, SCHUNK), lambda b, t, ids: (b, 0, t)),
                pl.BlockSpec((1, CHM, SCHUNK), lambda b, t, ids: (b, 0, t)),
            ],
            out_specs=[
                pl.BlockSpec((1, 1, 128), lambda b, t, ids: (b, 0, 0)),
                pl.BlockSpec((1, 128), lambda b, t, ids: (0, 0)),
            ],
            scratch_shapes=[
                pltpu.VMEM((8, 128), jnp.int32),
                pltpu.VMEM((8, 128), jnp.float32),
                pltpu.VMEM((128, 128), jnp.int32),
                pltpu.VMEM((NTGT, 4, SCHUNK), jnp.float32),
                pltpu.VMEM((8, SCHUNK), jnp.float32),
            ],
        ),
        out_shape=[
            jax.ShapeDtypeStruct((BSZ, 1, 128), jnp.int32),
            jax.ShapeDtypeStruct((1, 128), jnp.float32),
        ],
    )(tgt_ids, hmrel4, toff, sub4, obj4, hmrel3, hm3, thm3)


def _sc_gather_l1(tab_a, tab_b, idx_a, idx_b, tgt_a, tgt_b, msk):
    n = idx_a.shape[0] * idx_a.shape[1]
    mesh = plsc.VectorSubcoreMesh(core_axis_name="c", subcore_axis_name="s")

    @functools.partial(
        pl.kernel, mesh=mesh,
        out_type=jax.ShapeDtypeStruct((8, 16), jnp.float32),
        scratch_types=[
            pltpu.VMEM(idx_a.shape, jnp.int32),
            pltpu.VMEM(idx_a.shape, jnp.int32),
            pltpu.VMEM((n,), jnp.float32),
            pltpu.VMEM((n,), jnp.float32),
            pltpu.VMEM((n,), jnp.float32),
            pltpu.VMEM((n,), jnp.float32),
            pltpu.VMEM((n,), jnp.float32),
            pltpu.VMEM((8, 16), jnp.float32),
            pltpu.SemaphoreType.DMA,
        ],
    )
    def sck(ta_hbm, tb_hbm, ia_hbm, ib_hbm, tga_hbm, tgb_hbm, m_hbm, out_hbm,
            ia_v, ib_v, ga_v, gb_v, tga_v, tgb_v, m_v, o_v, sem):
        wid = lax.axis_index("s") * 2 + lax.axis_index("c")

        @pl.when(wid == 0)
        def _():
            pltpu.sync_copy(ia_hbm, ia_v)
            pltpu.sync_copy(ib_hbm, ib_v)
            pltpu.sync_copy(tga_hbm, tga_v)
            pltpu.sync_copy(tgb_hbm, tgb_v)
            pltpu.sync_copy(m_hbm, m_v)
            nrows = idx_a.shape[0]
            cps = []
            for r in range(nrows):
                cps.append(pltpu.async_copy(
                    ta_hbm.at[ia_v.at[r]], ga_v.at[pl.ds(r * 128, 128)], sem))
                cps.append(pltpu.async_copy(
                    tb_hbm.at[ib_v.at[r]], gb_v.at[pl.ds(r * 128, 128)], sem))
            for cp in cps:
                cp.wait()
            za = jnp.zeros((16,), jnp.float32)
            zb = jnp.zeros((16,), jnp.float32)
            zm = jnp.zeros((16,), jnp.float32)
            zd = jnp.zeros((16,), jnp.float32)
            for i in range(n // 16):
                sl = pl.ds(i * 16, 16)
                mv = m_v[sl]
                ga = ga_v[sl]
                gb = gb_v[sl]
                za = za + jnp.abs(ga * mv - tga_v[sl] * mv)
                zb = zb + jnp.abs(gb * mv - tgb_v[sl] * mv)
                zm = zm + mv
                zd = zd + jnp.abs(ga - gb)
            o_v[0] = za
            o_v[1] = zb
            o_v[2] = zm
            o_v[3] = zd
            zz = jnp.zeros((16,), jnp.float32)
            for r in range(4, 8):
                o_v[r] = zz
            pltpu.sync_copy(o_v, out_hbm)

    return sck(tab_a, tab_b, idx_a, idx_b, tgt_a, tgt_b, msk)


def _sc_offset_l1(sub_t, obj_t, tgt_t, idx_so, idx_t1, idx_t2):
    n = idx_so.shape[0] * idx_so.shape[1]
    mesh = plsc.VectorSubcoreMesh(core_axis_name="c", subcore_axis_name="s")

    @functools.partial(
        pl.kernel, mesh=mesh,
        out_type=jax.ShapeDtypeStruct((8, 16), jnp.float32),
        scratch_types=[
            pltpu.VMEM(idx_so.shape, jnp.int32),
            pltpu.VMEM(idx_so.shape, jnp.int32),
            pltpu.VMEM(idx_so.shape, jnp.int32),
            pltpu.VMEM((n,), jnp.float32),
            pltpu.VMEM((n,), jnp.float32),
            pltpu.VMEM((n,), jnp.float32),
            pltpu.VMEM((n,), jnp.float32),
            pltpu.VMEM((8, 16), jnp.float32),
            pltpu.SemaphoreType.DMA,
        ],
    )
    def sck(sub_hbm, obj_hbm, tgt_hbm, iso_hbm, it1_hbm, it2_hbm, out_hbm,
            iso_v, it1_v, it2_v, gs_v, go_v, g1_v, g2_v, o_v, sem):
        wid = lax.axis_index("s") * 2 + lax.axis_index("c")

        @pl.when(wid == 0)
        def _():
            pltpu.sync_copy(iso_hbm, iso_v)
            pltpu.sync_copy(it1_hbm, it1_v)
            pltpu.sync_copy(it2_hbm, it2_v)
            cps = []
            for r in range(idx_so.shape[0]):
                dst = pl.ds(r * 128, 128)
                cps.append(pltpu.async_copy(sub_hbm.at[iso_v.at[r]],
                                            gs_v.at[dst], sem))
                cps.append(pltpu.async_copy(obj_hbm.at[iso_v.at[r]],
                                            go_v.at[dst], sem))
                cps.append(pltpu.async_copy(tgt_hbm.at[it1_v.at[r]],
                                            g1_v.at[dst], sem))
                cps.append(pltpu.async_copy(tgt_hbm.at[it2_v.at[r]],
                                            g2_v.at[dst], sem))
            for cp in cps:
                cp.wait()
            zz = jnp.zeros((16,), jnp.float32)
            zd = zz
            for i in range(n // 16):
                sl = pl.ds(i * 16, 16)
                zd = (zd + jnp.abs(gs_v[sl] - g1_v[sl])
                      + jnp.abs(go_v[sl] - g2_v[sl]))
            o_v[0] = zd
            for r in range(1, 8):
                o_v[r] = zz
            pltpu.sync_copy(o_v, out_hbm)

    return sck(sub_t, obj_t, tgt_t, idx_so, idx_t1, idx_t2)


def kernel(out_hm_rel, out_sub_offset, out_obj_offset, out_hm, out_wh,
           out_reg, tgt_hm_rel, tgt_offset, tgt_offset_mask, tgt_hm,
           tgt_reg_mask, tgt_ind, tgt_wh, tgt_reg):
    f32 = jnp.float32
    tgt_ids = tgt_hm_rel.astype(jnp.int32)
    hmrel4 = out_hm_rel.reshape(BSZ, KCLS, SCHUNK, NTGT)
    hmrel3 = out_hm_rel.reshape(BSZ, KCLS, HWF)
    toff = tgt_offset.reshape(BSZ, NTGT, 4, HWF)
    sub4 = out_sub_offset.reshape(BSZ, 2, SCHUNK, NTGT)
    obj4 = out_obj_offset.reshape(BSZ, 2, SCHUNK, NTGT)
    hm3 = out_hm.reshape(BSZ, CHM, HWF)
    thm3 = tgt_hm.reshape(BSZ, CHM, HWF)

    ind = tgt_ind.astype(jnp.int32)
    bb = jnp.arange(BSZ, dtype=jnp.int32)[:, None, None]
    cc2 = jnp.arange(2, dtype=jnp.int32)[None, None, :]
    idx_wr = ((bb * 2 + cc2) * HWF + ind[:, :, None]).reshape(4, 128)
    tw = tgt_wh.astype(f32).reshape(-1)
    tr = tgt_reg.astype(f32).reshape(-1)
    mexp = jnp.broadcast_to(tgt_reg_mask[:, :, None].astype(f32),
                            (BSZ, MAXO, 2)).reshape(-1)
    sums1 = _sc_gather_l1(out_wh.reshape(-1), out_reg.reshape(-1),
                          idx_wr, idx_wr, tw, tr, mexp)

    src_arr, acc = _run_tc(tgt_ids, hmrel4, toff, sub4, obj4, hmrel3, hm3,
                           thm3)
    f0sum = acc[0, 0]
    posl = acc[0, 1]
    negl = acc[0, 2]
    npos = acc[0, 3]
    hm_rel_loss = f0sum
    hm_loss = jnp.where(npos > 0, -(posl + negl) / jnp.maximum(npos, 1.0),
                        -negl)

    src2 = src_arr[:, 0, :NTGT]
    mm = jnp.arange(NTGT, dtype=jnp.int32)[None, :, None]
    bb2 = jnp.arange(BSZ, dtype=jnp.int32)[:, None, None]
    idx_so = ((bb2 * 2 + cc2) * HWF + src2[:, :, None]).reshape(2, 128)
    idx_t1 = (((bb2 * NTGT + mm) * 4 + cc2) * HWF
              + src2[:, :, None]).reshape(2, 128)
    idx_t2 = (((bb2 * NTGT + mm) * 4 + cc2 + 2) * HWF
              + src2[:, :, None]).reshape(2, 128)

    wh_sum = jnp.sum(sums1[0])
    reg_sum = jnp.sum(sums1[1])
    msum = jnp.sum(sums1[2])
    wh_loss = wh_sum / (msum + 1e-4)
    off_loss = reg_sum / (msum + 1e-4)

    sums_off = _sc_offset_l1(out_sub_offset.reshape(-1),
                             out_obj_offset.reshape(-1),
                             tgt_offset.reshape(-1),
                             idx_so, idx_t1, idx_t2)
    offset_loss = jnp.sum(sums_off[0]) / float(BSZ * NTGT)

    loss = (1.0 * (hm_loss + hm_rel_loss)
            + 0.1 * (wh_loss + offset_loss)
            + 1.0 * off_loss)
    return (loss, hm_loss, wh_loss, off_loss, hm_rel_loss, offset_loss)


# speedup vs baseline: 1.8425x; 1.0939x over previous
import functools

import jax
import jax.numpy as jnp
from jax import lax
from jax.experimental import pallas as pl
from jax.experimental.pallas import tpu as pltpu
from jax.experimental.pallas import tpu_sc as plsc

BSZ, KCLS, HH, WW = 2, 117, 128, 128
HWF = HH * WW
NTGT = 64
MAXO = 128
CHM = 80
ALPHA_ = 0.25
SCHUNK = HWF // NTGT


def _softplus(x):
    return jnp.maximum(x, 0.0) + jnp.log1p(jnp.exp(-jnp.abs(x)))


def _tc_body(ids_ref, cmapa_ref, cmapb_ref, toff_ref,
             sub_ref, obj_ref, hr_ref, hm_ref, thm_ref,
             src_out, acc_out, srcs_scr, logit_scr, smatt_scr, oot_scr,
             acc_v):
    b = pl.program_id(0)
    u = pl.program_id(1)

    @pl.when((b == 0) & (u == 0))
    def _init_acc():
        acc_v[...] = jnp.zeros((8, 2 * SCHUNK), jnp.float32)

    @pl.when(u == 0)
    def _init_rows():
        srcs_scr[0:1, :] = jnp.zeros((1, 128), jnp.int32)
        logit_scr[0:1, :] = jnp.zeros((1, 128), jnp.float32)
        oot_scr[:, 0, 0:SCHUNK] = sub_ref[0, 0].T
        oot_scr[:, 1, 0:SCHUNK] = sub_ref[0, 1].T
        oot_scr[:, 2, 0:SCHUNK] = obj_ref[0, 0].T
        oot_scr[:, 3, 0:SCHUNK] = obj_ref[0, 1].T
        oot_scr[:, :, SCHUNK:] = oot_scr[:, :, 0:SCHUNK]

    x = hr_ref[0]
    ex = jnp.exp(-jnp.abs(x))
    rx = 1.0 / (1.0 + ex)
    px = jnp.where(x >= 0, rx, 1.0 - rx)
    f0 = ((1.0 - ALPHA_) * px * px
          * (jnp.maximum(x, 0.0) + jnp.log1p(ex)))

    y = hm_ref[0]
    ey = jnp.exp(-jnp.abs(y))
    ry = 1.0 / (1.0 + ey)
    sy = jnp.where(y >= 0, ry, 1.0 - ry)
    pr = jnp.clip(sy, 1e-4, 1.0 - 1e-4)
    spy = jnp.log1p(ey)
    lcl = -9.210340371976182
    lch = -1.0000500033334732e-04
    logp = jnp.clip(jnp.minimum(y, 0.0) - spy, lcl, lch)
    log1mp = jnp.clip(-jnp.maximum(y, 0.0) - spy, lcl, lch)
    gt = thm_ref[0]
    pos = (gt == 1.0).astype(jnp.float32)
    neg = (gt < 1.0).astype(jnp.float32)
    onemg = 1.0 - gt
    negw = (onemg * onemg) * (onemg * onemg)
    posl = logp * (1.0 - pr) * (1.0 - pr) * pos
    negl = log1mp * pr * pr * negw * neg

    acc_v[0:1, :] = (acc_v[0:1, :] + jnp.sum(f0, axis=0, keepdims=True))
    acc_v[1:2, :] = (acc_v[1:2, :] + jnp.sum(posl, axis=0, keepdims=True))
    acc_v[2:3, :] = (acc_v[2:3, :] + jnp.sum(negl, axis=0, keepdims=True))
    acc_v[3:4, :] = (acc_v[3:4, :] + jnp.sum(pos, axis=0, keepdims=True))

    diff = jnp.abs(toff_ref[0] - oot_scr[...])
    diff2 = diff.reshape(4 * NTGT, 2 * SCHUNK)
    rr = lax.broadcasted_iota(jnp.int32, (NTGT, 4 * NTGT), 1)
    nn2 = lax.broadcasted_iota(jnp.int32, (NTGT, 4 * NTGT), 0)
    sel = ((rr >> 2) == nn2).astype(jnp.float32)
    co2 = lax.dot_general(sel, diff2, (((1,), (0,)), ((), ())),
                          preferred_element_type=jnp.float32)

    ns = lax.broadcasted_iota(jnp.int32, (NTGT, SCHUNK), 0)
    ss = lax.broadcasted_iota(jnp.int32, (NTGT, SCHUNK), 1)
    pm = ss * NTGT + ns
    lane = lax.broadcasted_iota(jnp.int32, (1, 128), 1)

    for half, cm_ref in ((0, cmapa_ref), (1, cmapb_ref)):
        t = 2 * u + half
        cm = cm_ref[0, 0]
        cmT = cm.T
        ec = jnp.exp(-jnp.abs(cmT))
        rc = 1.0 / (1.0 + ec)
        prb = jnp.where(cmT >= 0, rc, 1.0 - rc)
        spc = jnp.log1p(ec)
        posc = (ALPHA_ * (1.0 - prb) * (1.0 - prb)
                * (jnp.maximum(-cmT, 0.0) + spc))
        negc = (1.0 - ALPHA_) * prb * prb * (jnp.maximum(cmT, 0.0) + spc)
        cc = posc - negc

        col = 0.1 * co2[:, half * SCHUNK:(half + 1) * SCHUNK] + cc
        mn = jnp.min(col)
        idx = jnp.min(jnp.where(col == mn, pm, jnp.int32(1 << 30)))
        logit = jnp.sum(jnp.where(pm == idx, cmT, 0.0))

        srcs_scr[0:1, :] = jnp.where(lane == t, idx, srcs_scr[0:1, :])
        logit_scr[0:1, :] = jnp.where(lane == t, logit, logit_scr[0:1, :])
        smatt_scr[pl.ds(t, 1), :] = jnp.full((1, 128), idx, jnp.int32)

    @pl.when(u == NTGT // 2 - 1)
    def _finish_batch():
        srow = srcs_scr[0:1, :]
        lrow = logit_scr[0:1, :]
        smat = jnp.broadcast_to(srow, (128, 128))
        smat_t = smatt_scr[...]
        ii = lax.broadcasted_iota(jnp.int32, (128, 128), 0)
        jj = lax.broadcasted_iota(jnp.int32, (128, 128), 1)
        later = ((smat == smat_t) & (ii > jj) & (ii < NTGT)).astype(jnp.int32)
        dead = jnp.max(later, axis=0, keepdims=True)
        keep = (dead == 0) & (lane < NTGT)
        sg = jax.nn.sigmoid(lrow)
        sp_p = _softplus(lrow)
        sp_n = sp_p - lrow
        corr = (ALPHA_ * (1.0 - sg) * (1.0 - sg) * sp_n
                - (1.0 - ALPHA_) * sg * sg * sp_p)
        acc_v[4:5, 0:128] = (acc_v[4:5, 0:128]
                             + jnp.where(keep, corr, 0.0))
        src_out[0] = jnp.where(lane < NTGT, srow, 0)

    @pl.when((b == BSZ - 1) & (u == NTGT // 2 - 1))
    def _emit_acc():
        s0 = jnp.sum(acc_v[0:1, :]) + jnp.sum(acc_v[4:5, :])
        s1 = jnp.sum(acc_v[1:2, :])
        s2 = jnp.sum(acc_v[2:3, :])
        s3 = jnp.sum(acc_v[3:4, :])
        v = (jnp.where(lane == 0, s0, 0.0)
             + jnp.where(lane == 1, s1, 0.0)
             + jnp.where(lane == 2, s2, 0.0)
             + jnp.where(lane == 3, s3, 0.0))
        acc_out[...] = v


def _run_tc(tgt_ids, hmrel4, toff, sub4, obj4, hmrel3, hm3, thm3):
    grid = (BSZ, NTGT // 2)

    return pl.pallas_call(
        _tc_body,
        grid_spec=pltpu.PrefetchScalarGridSpec(
            num_scalar_prefetch=1,
            grid=grid,
            in_specs=[
                pl.BlockSpec((1, 1, SCHUNK, NTGT),
                             lambda b, u, ids: (b, ids[b, 2 * u], 0, 0)),
                pl.BlockSpec((1, 1, SCHUNK, NTGT),
                             lambda b, u, ids: (b, ids[b, 2 * u + 1], 0, 0)),
                pl.BlockSpec((1, NTGT, 4, 2 * SCHUNK),
                             lambda b, u, ids: (b, 0, 0, u)),
                pl.BlockSpec((1, 2, SCHUNK, NTGT),
                             lambda b, u, ids: (b, 0, 0, 0)),
                pl.BlockSpec((1, 2, SCHUNK, NTGT),
                             lambda b, u, ids: (b, 0, 0, 0)),
                pl.BlockSpec((1, KCLS, 2 * SCHUNK),
                             lambda b, u, ids: (b, 0, u)),
                pl.BlockSpec((1, CHM, 2 * SCHUNK),
                             lambda b, u, ids: (b, 0, u)),
                pl.BlockSpec((1, CHM, 2 * SCHUNK),
                             lambda b, u, ids: (b, 0, u)),
            ],
            out_specs=[
                pl.BlockSpec((1, 1, 128), lambda b, u, ids: (b, 0, 0)),
                pl.BlockSpec((1, 128), lambda b, u, ids: (0, 0)),
            ],
            scratch_shapes=[
                pltpu.VMEM((8, 128), jnp.int32),
                pltpu.VMEM((8, 128), jnp.float32),
                pltpu.VMEM((128, 128), jnp.int32),
                pltpu.VMEM((NTGT, 4, 2 * SCHUNK), jnp.float32),
                pltpu.VMEM((8, 2 * SCHUNK), jnp.float32),
            ],
        ),
        out_shape=[
            jax.ShapeDtypeStruct((BSZ, 1, 128), jnp.int32),
            jax.ShapeDtypeStruct((1, 128), jnp.float32),
        ],
    )(tgt_ids, hmrel4, hmrel4, toff, sub4, obj4, hmrel3, hm3, thm3)


def _sc_gather_l1(tab_a, tab_b, idx_a, idx_b, tgt_a, tgt_b, msk):
    n = idx_a.shape[0] * idx_a.shape[1]
    mesh = plsc.VectorSubcoreMesh(core_axis_name="c", subcore_axis_name="s")

    @functools.partial(
        pl.kernel, mesh=mesh,
        out_type=jax.ShapeDtypeStruct((8, 16), jnp.float32),
        scratch_types=[
            pltpu.VMEM(idx_a.shape, jnp.int32),
            pltpu.VMEM(idx_a.shape, jnp.int32),
            pltpu.VMEM((n,), jnp.float32),
            pltpu.VMEM((n,), jnp.float32),
            pltpu.VMEM((n,), jnp.float32),
            pltpu.VMEM((n,), jnp.float32),
            pltpu.VMEM((n,), jnp.float32),
            pltpu.VMEM((8, 16), jnp.float32),
            pltpu.SemaphoreType.DMA,
        ],
    )
    def sck(ta_hbm, tb_hbm, ia_hbm, ib_hbm, tga_hbm, tgb_hbm, m_hbm, out_hbm,
            ia_v, ib_v, ga_v, gb_v, tga_v, tgb_v, m_v, o_v, sem):
        wid = lax.axis_index("s") * 2 + lax.axis_index("c")

        @pl.when(wid == 0)
        def _():
            pltpu.sync_copy(ia_hbm, ia_v)
            pltpu.sync_copy(ib_hbm, ib_v)
            pltpu.sync_copy(tga_hbm, tga_v)
            pltpu.sync_copy(tgb_hbm, tgb_v)
            pltpu.sync_copy(m_hbm, m_v)
            nrows = idx_a.shape[0]
            cps = []
            for r in range(nrows):
                cps.append(pltpu.async_copy(
                    ta_hbm.at[ia_v.at[r]], ga_v.at[pl.ds(r * 128, 128)], sem))
                cps.append(pltpu.async_copy(
                    tb_hbm.at[ib_v.at[r]], gb_v.at[pl.ds(r * 128, 128)], sem))
            for cp in cps:
                cp.wait()
            za = jnp.zeros((16,), jnp.float32)
            zb = jnp.zeros((16,), jnp.float32)
            zm = jnp.zeros((16,), jnp.float32)
            zd = jnp.zeros((16,), jnp.float32)
            for i in range(n // 16):
                sl = pl.ds(i * 16, 16)
                mv = m_v[sl]
                ga = ga_v[sl]
                gb = gb_v[sl]
                za = za + jnp.abs(ga * mv - tga_v[sl] * mv)
                zb = zb + jnp.abs(gb * mv - tgb_v[sl] * mv)
                zm = zm + mv
                zd = zd + jnp.abs(ga - gb)
            o_v[0] = za
            o_v[1] = zb
            o_v[2] = zm
            o_v[3] = zd
            zz = jnp.zeros((16,), jnp.float32)
            for r in range(4, 8):
                o_v[r] = zz
            pltpu.sync_copy(o_v, out_hbm)

    return sck(tab_a, tab_b, idx_a, idx_b, tgt_a, tgt_b, msk)


def _sc_offset_l1(sub_t, obj_t, tgt_t, idx_so, idx_t1, idx_t2):
    n = idx_so.shape[0] * idx_so.shape[1]
    mesh = plsc.VectorSubcoreMesh(core_axis_name="c", subcore_axis_name="s")

    @functools.partial(
        pl.kernel, mesh=mesh,
        out_type=jax.ShapeDtypeStruct((8, 16), jnp.float32),
        scratch_types=[
            pltpu.VMEM(idx_so.shape, jnp.int32),
            pltpu.VMEM(idx_so.shape, jnp.int32),
            pltpu.VMEM(idx_so.shape, jnp.int32),
            pltpu.VMEM((n,), jnp.float32),
            pltpu.VMEM((n,), jnp.float32),
            pltpu.VMEM((n,), jnp.float32),
            pltpu.VMEM((n,), jnp.float32),
            pltpu.VMEM((8, 16), jnp.float32),
            pltpu.SemaphoreType.DMA,
        ],
    )
    def sck(sub_hbm, obj_hbm, tgt_hbm, iso_hbm, it1_hbm, it2_hbm, out_hbm,
            iso_v, it1_v, it2_v, gs_v, go_v, g1_v, g2_v, o_v, sem):
        wid = lax.axis_index("s") * 2 + lax.axis_index("c")

        @pl.when(wid == 0)
        def _():
            pltpu.sync_copy(iso_hbm, iso_v)
            pltpu.sync_copy(it1_hbm, it1_v)
            pltpu.sync_copy(it2_hbm, it2_v)
            cps = []
            for r in range(idx_so.shape[0]):
                dst = pl.ds(r * 128, 128)
                cps.append(pltpu.async_copy(sub_hbm.at[iso_v.at[r]],
                                            gs_v.at[dst], sem))
                cps.append(pltpu.async_copy(obj_hbm.at[iso_v.at[r]],
                                            go_v.at[dst], sem))
                cps.append(pltpu.async_copy(tgt_hbm.at[it1_v.at[r]],
                                            g1_v.at[dst], sem))
                cps.append(pltpu.async_copy(tgt_hbm.at[it2_v.at[r]],
                                            g2_v.at[dst], sem))
            for cp in cps:
                cp.wait()
            zz = jnp.zeros((16,), jnp.float32)
            zd = zz
            for i in range(n // 16):
                sl = pl.ds(i * 16, 16)
                zd = (zd + jnp.abs(gs_v[sl] - g1_v[sl])
                      + jnp.abs(go_v[sl] - g2_v[sl]))
            o_v[0] = zd
            for r in range(1, 8):
                o_v[r] = zz
            pltpu.sync_copy(o_v, out_hbm)

    return sck(sub_t, obj_t, tgt_t, idx_so, idx_t1, idx_t2)


def kernel(out_hm_rel, out_sub_offset, out_obj_offset, out_hm, out_wh,
           out_reg, tgt_hm_rel, tgt_offset, tgt_offset_mask, tgt_hm,
           tgt_reg_mask, tgt_ind, tgt_wh, tgt_reg):
    f32 = jnp.float32
    tgt_ids = tgt_hm_rel.astype(jnp.int32)
    hmrel4 = out_hm_rel.reshape(BSZ, KCLS, SCHUNK, NTGT)
    hmrel3 = out_hm_rel.reshape(BSZ, KCLS, HWF)
    toff = tgt_offset.reshape(BSZ, NTGT, 4, HWF)
    sub4 = out_sub_offset.reshape(BSZ, 2, SCHUNK, NTGT)
    obj4 = out_obj_offset.reshape(BSZ, 2, SCHUNK, NTGT)
    hm3 = out_hm.reshape(BSZ, CHM, HWF)
    thm3 = tgt_hm.reshape(BSZ, CHM, HWF)

    ind = tgt_ind.astype(jnp.int32)
    bb = jnp.arange(BSZ, dtype=jnp.int32)[:, None, None]
    cc2 = jnp.arange(2, dtype=jnp.int32)[None, None, :]
    idx_wr = ((bb * 2 + cc2) * HWF + ind[:, :, None]).reshape(4, 128)
    tw = tgt_wh.astype(f32).reshape(-1)
    tr = tgt_reg.astype(f32).reshape(-1)
    mexp = jnp.broadcast_to(tgt_reg_mask[:, :, None].astype(f32),
                            (BSZ, MAXO, 2)).reshape(-1)
    sums1 = _sc_gather_l1(out_wh.reshape(-1), out_reg.reshape(-1),
                          idx_wr, idx_wr, tw, tr, mexp)

    src_arr, acc = _run_tc(tgt_ids, hmrel4, toff, sub4, obj4, hmrel3, hm3,
                           thm3)
    f0sum = acc[0, 0]
    posl = acc[0, 1]
    negl = acc[0, 2]
    npos = acc[0, 3]
    hm_rel_loss = f0sum
    hm_loss = jnp.where(npos > 0, -(posl + negl) / jnp.maximum(npos, 1.0),
                        -negl)

    src2 = src_arr[:, 0, :NTGT]
    mm = jnp.arange(NTGT, dtype=jnp.int32)[None, :, None]
    bb2 = jnp.arange(BSZ, dtype=jnp.int32)[:, None, None]
    idx_so = ((bb2 * 2 + cc2) * HWF + src2[:, :, None]).reshape(2, 128)
    idx_t1 = (((bb2 * NTGT + mm) * 4 + cc2) * HWF
              + src2[:, :, None]).reshape(2, 128)
    idx_t2 = (((bb2 * NTGT + mm) * 4 + cc2 + 2) * HWF
              + src2[:, :, None]).reshape(2, 128)

    wh_sum = jnp.sum(sums1[0])
    reg_sum = jnp.sum(sums1[1])
    msum = jnp.sum(sums1[2])
    wh_loss = wh_sum / (msum + 1e-4)
    off_loss = reg_sum / (msum + 1e-4)

    sums_off = _sc_offset_l1(out_sub_offset.reshape(-1),
                             out_obj_offset.reshape(-1),
                             tgt_offset.reshape(-1),
                             idx_so, idx_t1, idx_t2)
    offset_loss = jnp.sum(sums_off[0]) / float(BSZ * NTGT)

    loss = (1.0 * (hm_loss + hm_rel_loss)
            + 0.1 * (wh_loss + offset_loss)
            + 1.0 * off_loss)
    return (loss, hm_loss, wh_loss, off_loss, hm_rel_loss, offset_loss)


# speedup vs baseline: 1.8813x; 1.0210x over previous
import functools

import jax
import jax.numpy as jnp
from jax import lax
from jax.experimental import pallas as pl
from jax.experimental.pallas import tpu as pltpu
from jax.experimental.pallas import tpu_sc as plsc

BSZ, KCLS, HH, WW = 2, 117, 128, 128
HWF = HH * WW
NTGT = 64
MAXO = 128
CHM = 80
ALPHA_ = 0.25
SCHUNK = HWF // NTGT
NPER = 4
WCH = NPER * SCHUNK


def _softplus(x):
    return jnp.maximum(x, 0.0) + jnp.log1p(jnp.exp(-jnp.abs(x)))


def _tc_body(ids_ref, *refs):
    cmap_refs = refs[:NPER]
    (toff_ref, sub_ref, obj_ref, hr_ref, hm_ref, thm_ref,
     src_out, acc_out, srcs_scr, logit_scr, smatt_scr, oot_scr,
     acc_v) = refs[NPER:]
    b = pl.program_id(0)
    u = pl.program_id(1)

    @pl.when((b == 0) & (u == 0))
    def _init_acc():
        acc_v[...] = jnp.zeros((8, WCH), jnp.float32)

    @pl.when(u == 0)
    def _init_rows():
        srcs_scr[0:1, :] = jnp.zeros((1, 128), jnp.int32)
        logit_scr[0:1, :] = jnp.zeros((1, 128), jnp.float32)
        oot_scr[:, 0, 0:SCHUNK] = sub_ref[0, 0].T
        oot_scr[:, 1, 0:SCHUNK] = sub_ref[0, 1].T
        oot_scr[:, 2, 0:SCHUNK] = obj_ref[0, 0].T
        oot_scr[:, 3, 0:SCHUNK] = obj_ref[0, 1].T
        for h in range(1, NPER):
            oot_scr[:, :, h * SCHUNK:(h + 1) * SCHUNK] = (
                oot_scr[:, :, 0:SCHUNK])

    x = hr_ref[0]
    ex = jnp.exp(-jnp.abs(x))
    rx = 1.0 / (1.0 + ex)
    px = jnp.where(x >= 0, rx, 1.0 - rx)
    f0 = ((1.0 - ALPHA_) * px * px
          * (jnp.maximum(x, 0.0) + jnp.log1p(ex)))

    y = hm_ref[0]
    ey = jnp.exp(-jnp.abs(y))
    ry = 1.0 / (1.0 + ey)
    sy = jnp.where(y >= 0, ry, 1.0 - ry)
    pr = jnp.clip(sy, 1e-4, 1.0 - 1e-4)
    spy = jnp.log1p(ey)
    lcl = -9.210340371976182
    lch = -1.0000500033334732e-04
    logp = jnp.clip(jnp.minimum(y, 0.0) - spy, lcl, lch)
    log1mp = jnp.clip(-jnp.maximum(y, 0.0) - spy, lcl, lch)
    gt = thm_ref[0]
    pos = (gt == 1.0).astype(jnp.float32)
    neg = (gt < 1.0).astype(jnp.float32)
    onemg = 1.0 - gt
    negw = (onemg * onemg) * (onemg * onemg)
    posl = logp * (1.0 - pr) * (1.0 - pr) * pos
    negl = log1mp * pr * pr * negw * neg

    acc_v[0:1, :] = (acc_v[0:1, :] + jnp.sum(f0, axis=0, keepdims=True))
    acc_v[1:2, :] = (acc_v[1:2, :] + jnp.sum(posl, axis=0, keepdims=True))
    acc_v[2:3, :] = (acc_v[2:3, :] + jnp.sum(negl, axis=0, keepdims=True))
    acc_v[3:4, :] = (acc_v[3:4, :] + jnp.sum(pos, axis=0, keepdims=True))

    diff = jnp.abs(toff_ref[0] - oot_scr[...])
    diff2 = diff.reshape(4 * NTGT, WCH)
    rr = lax.broadcasted_iota(jnp.int32, (NTGT, 4 * NTGT), 1)
    nn2 = lax.broadcasted_iota(jnp.int32, (NTGT, 4 * NTGT), 0)
    sel = ((rr >> 2) == nn2).astype(jnp.float32)
    co2 = lax.dot_general(sel, diff2, (((1,), (0,)), ((), ())),
                          preferred_element_type=jnp.float32)

    ns = lax.broadcasted_iota(jnp.int32, (NTGT, SCHUNK), 0)
    ss = lax.broadcasted_iota(jnp.int32, (NTGT, SCHUNK), 1)
    pm = ss * NTGT + ns
    lane = lax.broadcasted_iota(jnp.int32, (1, 128), 1)

    for half, cm_ref in enumerate(cmap_refs):
        t = NPER * u + half
        cm = cm_ref[0, 0]
        cmT = cm.T
        ec = jnp.exp(-jnp.abs(cmT))
        rc = 1.0 / (1.0 + ec)
        prb = jnp.where(cmT >= 0, rc, 1.0 - rc)
        spc = jnp.log1p(ec)
        posc = (ALPHA_ * (1.0 - prb) * (1.0 - prb)
                * (jnp.maximum(-cmT, 0.0) + spc))
        negc = (1.0 - ALPHA_) * prb * prb * (jnp.maximum(cmT, 0.0) + spc)
        cc = posc - negc

        col = 0.1 * co2[:, half * SCHUNK:(half + 1) * SCHUNK] + cc
        mn = jnp.min(col)
        idx = jnp.min(jnp.where(col == mn, pm, jnp.int32(1 << 30)))
        logit = jnp.sum(jnp.where(pm == idx, cmT, 0.0))

        srcs_scr[0:1, :] = jnp.where(lane == t, idx, srcs_scr[0:1, :])
        logit_scr[0:1, :] = jnp.where(lane == t, logit, logit_scr[0:1, :])
        smatt_scr[pl.ds(t, 1), :] = jnp.full((1, 128), idx, jnp.int32)

    @pl.when(u == NTGT // NPER - 1)
    def _finish_batch():
        srow = srcs_scr[0:1, :]
        lrow = logit_scr[0:1, :]
        smat = jnp.broadcast_to(srow, (128, 128))
        smat_t = smatt_scr[...]
        ii = lax.broadcasted_iota(jnp.int32, (128, 128), 0)
        jj = lax.broadcasted_iota(jnp.int32, (128, 128), 1)
        later = ((smat == smat_t) & (ii > jj) & (ii < NTGT)).astype(jnp.int32)
        dead = jnp.max(later, axis=0, keepdims=True)
        keep = (dead == 0) & (lane < NTGT)
        sg = jax.nn.sigmoid(lrow)
        sp_p = _softplus(lrow)
        sp_n = sp_p - lrow
        corr = (ALPHA_ * (1.0 - sg) * (1.0 - sg) * sp_n
                - (1.0 - ALPHA_) * sg * sg * sp_p)
        acc_v[4:5, 0:128] = (acc_v[4:5, 0:128]
                             + jnp.where(keep, corr, 0.0))
        src_out[0] = jnp.where(lane < NTGT, srow, 0)

    @pl.when((b == BSZ - 1) & (u == NTGT // NPER - 1))
    def _emit_acc():
        s0 = jnp.sum(acc_v[0:1, :]) + jnp.sum(acc_v[4:5, :])
        s1 = jnp.sum(acc_v[1:2, :])
        s2 = jnp.sum(acc_v[2:3, :])
        s3 = jnp.sum(acc_v[3:4, :])
        v = (jnp.where(lane == 0, s0, 0.0)
             + jnp.where(lane == 1, s1, 0.0)
             + jnp.where(lane == 2, s2, 0.0)
             + jnp.where(lane == 3, s3, 0.0))
        acc_out[...] = v


def _run_tc(tgt_ids, hmrel4, toff, sub4, obj4, hmrel3, hm3, thm3):
    grid = (BSZ, NTGT // NPER)

    def _cmap_spec(h):
        return pl.BlockSpec(
            (1, 1, SCHUNK, NTGT),
            lambda b, u, ids, _h=h: (b, ids[b, NPER * u + _h], 0, 0))

    return pl.pallas_call(
        _tc_body,
        grid_spec=pltpu.PrefetchScalarGridSpec(
            num_scalar_prefetch=1,
            grid=grid,
            in_specs=[
                *[_cmap_spec(h) for h in range(NPER)],
                pl.BlockSpec((1, NTGT, 4, WCH),
                             lambda b, u, ids: (b, 0, 0, u)),
                pl.BlockSpec((1, 2, SCHUNK, NTGT),
                             lambda b, u, ids: (b, 0, 0, 0)),
                pl.BlockSpec((1, 2, SCHUNK, NTGT),
                             lambda b, u, ids: (b, 0, 0, 0)),
                pl.BlockSpec((1, KCLS, WCH), lambda b, u, ids: (b, 0, u)),
                pl.BlockSpec((1, CHM, WCH), lambda b, u, ids: (b, 0, u)),
                pl.BlockSpec((1, CHM, WCH), lambda b, u, ids: (b, 0, u)),
            ],
            out_specs=[
                pl.BlockSpec((1, 1, 128), lambda b, u, ids: (b, 0, 0)),
                pl.BlockSpec((1, 128), lambda b, u, ids: (0, 0)),
            ],
            scratch_shapes=[
                pltpu.VMEM((8, 128), jnp.int32),
                pltpu.VMEM((8, 128), jnp.float32),
                pltpu.VMEM((128, 128), jnp.int32),
                pltpu.VMEM((NTGT, 4, WCH), jnp.float32),
                pltpu.VMEM((8, WCH), jnp.float32),
            ],
        ),
        out_shape=[
            jax.ShapeDtypeStruct((BSZ, 1, 128), jnp.int32),
            jax.ShapeDtypeStruct((1, 128), jnp.float32),
        ],
    )(tgt_ids, *([hmrel4] * NPER), toff, sub4, obj4, hmrel3, hm3, thm3)


def _sc_gather_l1(tab_a, tab_b, idx_a, idx_b, tgt_a, tgt_b, msk):
    n = idx_a.shape[0] * idx_a.shape[1]
    mesh = plsc.VectorSubcoreMesh(core_axis_name="c", subcore_axis_name="s")

    @functools.partial(
        pl.kernel, mesh=mesh,
        out_type=jax.ShapeDtypeStruct((8, 16), jnp.float32),
        scratch_types=[
            pltpu.VMEM(idx_a.shape, jnp.int32),
            pltpu.VMEM(idx_a.shape, jnp.int32),
            pltpu.VMEM((n,), jnp.float32),
            pltpu.VMEM((n,), jnp.float32),
            pltpu.VMEM((n,), jnp.float32),
            pltpu.VMEM((n,), jnp.float32),
            pltpu.VMEM((n,), jnp.float32),
            pltpu.VMEM((8, 16), jnp.float32),
            pltpu.SemaphoreType.DMA,
        ],
    )
    def sck(ta_hbm, tb_hbm, ia_hbm, ib_hbm, tga_hbm, tgb_hbm, m_hbm, out_hbm,
            ia_v, ib_v, ga_v, gb_v, tga_v, tgb_v, m_v, o_v, sem):
        wid = lax.axis_index("s") * 2 + lax.axis_index("c")

        @pl.when(wid == 0)
        def _():
            pltpu.sync_copy(ia_hbm, ia_v)
            pltpu.sync_copy(ib_hbm, ib_v)
            pltpu.sync_copy(tga_hbm, tga_v)
            pltpu.sync_copy(tgb_hbm, tgb_v)
            pltpu.sync_copy(m_hbm, m_v)
            nrows = idx_a.shape[0]
            cps = []
            for r in range(nrows):
                cps.append(pltpu.async_copy(
                    ta_hbm.at[ia_v.at[r]], ga_v.at[pl.ds(r * 128, 128)], sem))
                cps.append(pltpu.async_copy(
                    tb_hbm.at[ib_v.at[r]], gb_v.at[pl.ds(r * 128, 128)], sem))
            for cp in cps:
                cp.wait()
            za = jnp.zeros((16,), jnp.float32)
            zb = jnp.zeros((16,), jnp.float32)
            zm = jnp.zeros((16,), jnp.float32)
            zd = jnp.zeros((16,), jnp.float32)
            for i in range(n // 16):
                sl = pl.ds(i * 16, 16)
                mv = m_v[sl]
                ga = ga_v[sl]
                gb = gb_v[sl]
                za = za + jnp.abs(ga * mv - tga_v[sl] * mv)
                zb = zb + jnp.abs(gb * mv - tgb_v[sl] * mv)
                zm = zm + mv
                zd = zd + jnp.abs(ga - gb)
            o_v[0] = za
            o_v[1] = zb
            o_v[2] = zm
            o_v[3] = zd
            zz = jnp.zeros((16,), jnp.float32)
            for r in range(4, 8):
                o_v[r] = zz
            pltpu.sync_copy(o_v, out_hbm)

    return sck(tab_a, tab_b, idx_a, idx_b, tgt_a, tgt_b, msk)


def _sc_offset_l1(sub_t, obj_t, tgt_t, idx_so, idx_t1, idx_t2):
    n = idx_so.shape[0] * idx_so.shape[1]
    mesh = plsc.VectorSubcoreMesh(core_axis_name="c", subcore_axis_name="s")

    @functools.partial(
        pl.kernel, mesh=mesh,
        out_type=jax.ShapeDtypeStruct((8, 16), jnp.float32),
        scratch_types=[
            pltpu.VMEM(idx_so.shape, jnp.int32),
            pltpu.VMEM(idx_so.shape, jnp.int32),
            pltpu.VMEM(idx_so.shape, jnp.int32),
            pltpu.VMEM((n,), jnp.float32),
            pltpu.VMEM((n,), jnp.float32),
            pltpu.VMEM((n,), jnp.float32),
            pltpu.VMEM((n,), jnp.float32),
            pltpu.VMEM((8, 16), jnp.float32),
            pltpu.SemaphoreType.DMA,
        ],
    )
    def sck(sub_hbm, obj_hbm, tgt_hbm, iso_hbm, it1_hbm, it2_hbm, out_hbm,
            iso_v, it1_v, it2_v, gs_v, go_v, g1_v, g2_v, o_v, sem):
        wid = lax.axis_index("s") * 2 + lax.axis_index("c")

        @pl.when(wid == 0)
        def _():
            pltpu.sync_copy(iso_hbm, iso_v)
            pltpu.sync_copy(it1_hbm, it1_v)
            pltpu.sync_copy(it2_hbm, it2_v)
            cps = []
            for r in range(idx_so.shape[0]):
                dst = pl.ds(r * 128, 128)
                cps.append(pltpu.async_copy(sub_hbm.at[iso_v.at[r]],
                                            gs_v.at[dst], sem))
                cps.append(pltpu.async_copy(obj_hbm.at[iso_v.at[r]],
                                            go_v.at[dst], sem))
                cps.append(pltpu.async_copy(tgt_hbm.at[it1_v.at[r]],
                                            g1_v.at[dst], sem))
                cps.append(pltpu.async_copy(tgt_hbm.at[it2_v.at[r]],
                                            g2_v.at[dst], sem))
            for cp in cps:
                cp.wait()
            zz = jnp.zeros((16,), jnp.float32)
            zd = zz
            for i in range(n // 16):
                sl = pl.ds(i * 16, 16)
                zd = (zd + jnp.abs(gs_v[sl] - g1_v[sl])
                      + jnp.abs(go_v[sl] - g2_v[sl]))
            o_v[0] = zd
            for r in range(1, 8):
                o_v[r] = zz
            pltpu.sync_copy(o_v, out_hbm)

    return sck(sub_t, obj_t, tgt_t, idx_so, idx_t1, idx_t2)


def kernel(out_hm_rel, out_sub_offset, out_obj_offset, out_hm, out_wh,
           out_reg, tgt_hm_rel, tgt_offset, tgt_offset_mask, tgt_hm,
           tgt_reg_mask, tgt_ind, tgt_wh, tgt_reg):
    f32 = jnp.float32
    tgt_ids = tgt_hm_rel.astype(jnp.int32)
    hmrel4 = out_hm_rel.reshape(BSZ, KCLS, SCHUNK, NTGT)
    hmrel3 = out_hm_rel.reshape(BSZ, KCLS, HWF)
    toff = tgt_offset.reshape(BSZ, NTGT, 4, HWF)
    sub4 = out_sub_offset.reshape(BSZ, 2, SCHUNK, NTGT)
    obj4 = out_obj_offset.reshape(BSZ, 2, SCHUNK, NTGT)
    hm3 = out_hm.reshape(BSZ, CHM, HWF)
    thm3 = tgt_hm.reshape(BSZ, CHM, HWF)

    ind = tgt_ind.astype(jnp.int32)
    bb = jnp.arange(BSZ, dtype=jnp.int32)[:, None, None]
    cc2 = jnp.arange(2, dtype=jnp.int32)[None, None, :]
    idx_wr = ((bb * 2 + cc2) * HWF + ind[:, :, None]).reshape(4, 128)
    tw = tgt_wh.astype(f32).reshape(-1)
    tr = tgt_reg.astype(f32).reshape(-1)
    mexp = jnp.broadcast_to(tgt_reg_mask[:, :, None].astype(f32),
                            (BSZ, MAXO, 2)).reshape(-1)
    sums1 = _sc_gather_l1(out_wh.reshape(-1), out_reg.reshape(-1),
                          idx_wr, idx_wr, tw, tr, mexp)

    src_arr, acc = _run_tc(tgt_ids, hmrel4, toff, sub4, obj4, hmrel3, hm3,
                           thm3)
    f0sum = acc[0, 0]
    posl = acc[0, 1]
    negl = acc[0, 2]
    npos = acc[0, 3]
    hm_rel_loss = f0sum
    hm_loss = jnp.where(npos > 0, -(posl + negl) / jnp.maximum(npos, 1.0),
                        -negl)

    src2 = src_arr[:, 0, :NTGT]
    mm = jnp.arange(NTGT, dtype=jnp.int32)[None, :, None]
    bb2 = jnp.arange(BSZ, dtype=jnp.int32)[:, None, None]
    idx_so = ((bb2 * 2 + cc2) * HWF + src2[:, :, None]).reshape(2, 128)
    idx_t1 = (((bb2 * NTGT + mm) * 4 + cc2) * HWF
              + src2[:, :, None]).reshape(2, 128)
    idx_t2 = (((bb2 * NTGT + mm) * 4 + cc2 + 2) * HWF
              + src2[:, :, None]).reshape(2, 128)

    wh_sum = jnp.sum(sums1[0])
    reg_sum = jnp.sum(sums1[1])
    msum = jnp.sum(sums1[2])
    wh_loss = wh_sum / (msum + 1e-4)
    off_loss = reg_sum / (msum + 1e-4)

    sums_off = _sc_offset_l1(out_sub_offset.reshape(-1),
                             out_obj_offset.reshape(-1),
                             tgt_offset.reshape(-1),
                             idx_so, idx_t1, idx_t2)
    offset_loss = jnp.sum(sums_off[0]) / float(BSZ * NTGT)

    loss = (1.0 * (hm_loss + hm_rel_loss)
            + 0.1 * (wh_loss + offset_loss)
            + 1.0 * off_loss)
    return (loss, hm_loss, wh_loss, off_loss, hm_rel_loss, offset_loss)
